# baseline (speedup 1.0000x reference)
import functools

import jax
import jax.numpy as jnp
import numpy as np
from jax import lax
from jax.experimental import pallas as pl
from jax.experimental.pallas import tpu as pltpu
from jax.experimental.pallas import tpu_sc as plsc

N, E, D, H = 10000, 320000, 128, 64
NC, NS, L = 2, 16, 16
NW = NC * NS
CH = 128

EP = E + N
NCHUNK = 82
PER_W = NCHUNK * CH
E_PAD = NW * PER_W
E_IDX = E_PAD + 2 * CH

NCHUNK2 = 80
PER_W2 = NCHUNK2 * CH
E2_PAD = NW * PER_W2
E2_IDX = E2_PAD + 2 * CH

N_ACC = 10240
ROWS_PER_SUB = N_ACC // NS

@functools.cache
def _sc_mesh():
    return plsc.VectorSubcoreMesh(core_axis_name="c", subcore_axis_name="s",
                                  num_cores=NC, num_subcores=NS)


def _sc_conv_body(h_hbm, asrc_hbm, adst_hbm, gmax_hbm, src_hbm, dst_hbm,
                  out_hbm, den_hbm,
                  asrc_v, adst_v, gmax_v,
                  src_v0, dst_v0, rows_v0, src_v1, dst_v1, rows_v1, p_v,
                  zbuf, zden, acc_sh, den_sh,
                  gsem0, gsem1, si0, di0, si1, di1):
    cid = lax.axis_index("c")
    sid = lax.axis_index("s")
    wid = sid * NC + cid

    pltpu.sync_copy(asrc_hbm, asrc_v)
    pltpu.sync_copy(adst_hbm, adst_v)
    pltpu.sync_copy(gmax_hbm, gmax_v)

    def _zrow(i, _):
        for q in range(4):
            zbuf[i, pl.ds(q * L, L)] = jnp.zeros((L,), jnp.float32)
        return 0
    lax.fori_loop(0, CH, _zrow, 0)

    def _zden(i, _):
        zden[pl.ds(i * L, L)] = jnp.zeros((L,), jnp.float32)
        return 0
    lax.fori_loop(0, CH // L, _zden, 0)

    for t in range(ROWS_PER_SUB // CH):
        pltpu.sync_copy(zbuf, acc_sh.at[pl.ds(sid * ROWS_PER_SUB + t * CH, CH)])
        pltpu.sync_copy(zden, den_sh.at[pl.ds(sid * ROWS_PER_SUB + t * CH, CH)])
    plsc.subcore_barrier()

    base = wid * PER_W
    bufs = ((src_v0, dst_v0, rows_v0, gsem0, si0, di0),
            (src_v1, dst_v1, rows_v1, gsem1, si1, di1))

    def _process(k, cur, nxt):
        src_c, dst_c, rows_c, gsem_c, _, _ = cur
        src_n, dst_n, rows_n, gsem_n, si_n, di_n = nxt
        pltpu.make_async_copy(src_hbm.at[pl.ds(0, CH)], src_n, si_n).wait()
        pltpu.make_async_copy(dst_hbm.at[pl.ds(0, CH)], dst_n, di_n).wait()
        pltpu.async_copy(h_hbm.at[src_n], rows_n, gsem_n)
        cb = base + k * CH
        gvec = gmax_v[...]
        for g in range(CH // L):
            s_idx = src_c[pl.ds(g * L, L)]
            d_idx = dst_c[pl.ds(g * L, L)]
            a_s = plsc.load_gather(asrc_v, [s_idx])
            a_d = plsc.load_gather(adst_v, [d_idx])
            al = a_s + a_d
            al = jnp.where(al >= 0.0, al, 0.2 * al)
            m = gvec + a_d
            m = jnp.where(m >= 0.0, m, 0.2 * m)
            p = jnp.exp(al - m)
            pos = cb + g * L + lax.iota(jnp.int32, L)
            p = jnp.where(pos < EP, p, 0.0)
            p_v[pl.ds(g * L, L)] = p
        pltpu.make_async_copy(src_hbm.at[pl.ds(0, CH)], rows_c, gsem_c).wait()

        def _scale(j4, _):
            j = j4 * 4
            for u in range(4):
                pj = plsc.load_gather(p_v, [jnp.full((L,), j + u, jnp.int32)])
                for q in range(4):
                    rows_c[j + u, pl.ds(q * L, L)] = (
                        rows_c[j + u, pl.ds(q * L, L)] * pj)
            return 0
        lax.fori_loop(0, CH // 4, _scale, 0)
        pltpu.sync_copy(rows_c, acc_sh.at[dst_c], add=True)
        pltpu.sync_copy(p_v, den_sh.at[dst_c], add=True)
        nb = base + (k + 2) * CH
        pltpu.async_copy(src_hbm.at[pl.ds(nb, CH)], src_c, cur[4])
        pltpu.async_copy(dst_hbm.at[pl.ds(nb, CH)], dst_c, cur[5])

    pltpu.sync_copy(src_hbm.at[pl.ds(base, CH)], src_v0)
    pltpu.sync_copy(dst_hbm.at[pl.ds(base, CH)], dst_v0)
    pltpu.async_copy(src_hbm.at[pl.ds(base + CH, CH)], src_v1, si1)
    pltpu.async_copy(dst_hbm.at[pl.ds(base + CH, CH)], dst_v1, di1)
    pltpu.async_copy(h_hbm.at[src_v0], rows_v0, gsem0)

    def _pair_steps(t, _):
        _process(2 * t, bufs[0], bufs[1])
        _process(2 * t + 1, bufs[1], bufs[0])
        return 0
    lax.fori_loop(0, NCHUNK // 2, _pair_steps, 0)

    pltpu.make_async_copy(src_hbm.at[pl.ds(0, CH)], rows_v0, gsem0).wait()
    pltpu.make_async_copy(src_hbm.at[pl.ds(0, CH)], src_v1, si1).wait()
    pltpu.make_async_copy(dst_hbm.at[pl.ds(0, CH)], dst_v1, di1).wait()
    plsc.subcore_barrier()

    for t in range(ROWS_PER_SUB // CH):
        o = sid * ROWS_PER_SUB + t * CH
        pltpu.sync_copy(acc_sh.at[pl.ds(o, CH)], out_hbm.at[cid, pl.ds(o, CH)])
        pltpu.sync_copy(den_sh.at[pl.ds(o, CH)], den_hbm.at[cid, pl.ds(o, CH)])


@functools.cache
def _sc_conv_kernel():
  return pl.kernel(
    _sc_conv_body,
    out_type=(jax.ShapeDtypeStruct((NC, N_ACC, H), jnp.float32),
              jax.ShapeDtypeStruct((NC, N_ACC), jnp.float32)),
    mesh=_sc_mesh(),
    compiler_params=pltpu.CompilerParams(needs_layout_passes=False, use_tc_tiling_on_sc=False),
    scratch_types=[
        pltpu.VMEM((N,), jnp.float32),
        pltpu.VMEM((N,), jnp.float32),
        pltpu.VMEM((L,), jnp.float32),
        pltpu.VMEM((CH,), jnp.int32),
        pltpu.VMEM((CH,), jnp.int32),
        pltpu.VMEM((CH, H), jnp.float32),
        pltpu.VMEM((CH,), jnp.int32),
        pltpu.VMEM((CH,), jnp.int32),
        pltpu.VMEM((CH, H), jnp.float32),
        pltpu.VMEM((CH,), jnp.float32),
        pltpu.VMEM((CH, H), jnp.float32),
        pltpu.VMEM((CH,), jnp.float32),
        pltpu.VMEM_SHARED((N_ACC, H), jnp.float32),
        pltpu.VMEM_SHARED((N_ACC,), jnp.float32),
        pltpu.SemaphoreType.DMA,
        pltpu.SemaphoreType.DMA,
        pltpu.SemaphoreType.DMA,
        pltpu.SemaphoreType.DMA,
        pltpu.SemaphoreType.DMA,
        pltpu.SemaphoreType.DMA,
    ],
  )


def _sc_pair_body(p_hbm, q_hbm, row_hbm, col_hbm, s_hbm,
                  row_v0, col_v0, pbuf0, qbuf0, sbuf0,
                  row_v1, col_v1, pbuf1, qbuf1, sbuf1,
                  gp0, gq0, gp1, gq1, ri0, ci0, ri1, ci1, wsem0, wsem1):
    cid = lax.axis_index("c")
    sid = lax.axis_index("s")
    wid = sid * NC + cid
    base = wid * PER_W2
    bufs = ((row_v0, col_v0, pbuf0, qbuf0, sbuf0, gp0, gq0, ri0, ci0, wsem0),
            (row_v1, col_v1, pbuf1, qbuf1, sbuf1, gp1, gq1, ri1, ci1, wsem1))

    def _process(k, cur, nxt):
        row_c, col_c, pb_c, qb_c, sb_c, gp_c, gq_c, ri_c, ci_c, ws_c = cur
        row_n, col_n, pb_n, qb_n, sb_n, gp_n, gq_n, ri_n, ci_n, ws_n = nxt
        pltpu.make_async_copy(row_hbm.at[pl.ds(0, CH)], row_n, ri_n).wait()
        pltpu.make_async_copy(col_hbm.at[pl.ds(0, CH)], col_n, ci_n).wait()
        pltpu.async_copy(p_hbm.at[row_n], pb_n, gp_n)
        pltpu.async_copy(q_hbm.at[col_n], qb_n, gq_n)
        pltpu.make_async_copy(row_hbm.at[pl.ds(0, CH)], pb_c, gp_c).wait()
        pltpu.make_async_copy(row_hbm.at[pl.ds(0, CH)], qb_c, gq_c).wait()

        @pl.when(k >= 2)
        def _():
            pltpu.make_async_copy(row_hbm.at[pl.ds(0, CH)], sb_c, ws_c).wait()

        def _add(i, _):
            for hh in range(2):
                for q in range(4):
                    sb_c[i, pl.ds(hh * H + q * L, L)] = (
                        pb_c[2 * i + hh, pl.ds(q * L, L)]
                        + qb_c[2 * i + hh, pl.ds(q * L, L)])
            return 0
        lax.fori_loop(0, CH // 2, _add, 0)
        cb2 = (base + k * CH) // 2
        pltpu.async_copy(sb_c, s_hbm.at[pl.ds(cb2, CH // 2)], ws_c)
        nb = base + (k + 2) * CH
        pltpu.async_copy(row_hbm.at[pl.ds(nb, CH)], row_c, ri_c)
        pltpu.async_copy(col_hbm.at[pl.ds(nb, CH)], col_c, ci_c)

    pltpu.sync_copy(row_hbm.at[pl.ds(base, CH)], row_v0)
    pltpu.sync_copy(col_hbm.at[pl.ds(base, CH)], col_v0)
    pltpu.async_copy(row_hbm.at[pl.ds(base + CH, CH)], row_v1, ri1)
    pltpu.async_copy(col_hbm.at[pl.ds(base + CH, CH)], col_v1, ci1)
    pltpu.async_copy(p_hbm.at[row_v0], pbuf0, gp0)
    pltpu.async_copy(q_hbm.at[col_v0], qbuf0, gq0)

    def _pair_steps(t, _):
        _process(2 * t, bufs[0], bufs[1])
        _process(2 * t + 1, bufs[1], bufs[0])
        return 0
    lax.fori_loop(0, NCHUNK2 // 2, _pair_steps, 0)

    pltpu.make_async_copy(row_hbm.at[pl.ds(0, CH)], pbuf0, gp0).wait()
    pltpu.make_async_copy(row_hbm.at[pl.ds(0, CH)], qbuf0, gq0).wait()
    pltpu.make_async_copy(row_hbm.at[pl.ds(0, CH)], row_v1, ri1).wait()
    pltpu.make_async_copy(col_hbm.at[pl.ds(0, CH)], col_v1, ci1).wait()
    pltpu.make_async_copy(row_hbm.at[pl.ds(0, CH)], sbuf0, wsem0).wait()
    pltpu.make_async_copy(row_hbm.at[pl.ds(0, CH)], sbuf1, wsem1).wait()


@functools.cache
def _sc_pair_kernel():
  return pl.kernel(
    _sc_pair_body,
    out_type=jax.ShapeDtypeStruct((E2_PAD // 2, 2 * H), jnp.float32),
    mesh=_sc_mesh(),
    compiler_params=pltpu.CompilerParams(needs_layout_passes=False, use_tc_tiling_on_sc=False),
    scratch_types=(
        [pltpu.VMEM((CH,), jnp.int32), pltpu.VMEM((CH,), jnp.int32),
         pltpu.VMEM((CH, H), jnp.float32), pltpu.VMEM((CH, H), jnp.float32),
         pltpu.VMEM((CH // 2, 2 * H), jnp.float32)] * 2
        + [pltpu.SemaphoreType.DMA] * 10
    ),
  )


_BLK = 1000
_NBLK = N // _BLK


def _tc1_body(x_ref, w_ref, as_ref, ad_ref,
              h_ref, asrc_ref, adst_ref, gmax_ref):
    i = pl.program_id(0)
    h = jnp.dot(x_ref[...], w_ref[...], preferred_element_type=jnp.float32)
    h_ref[...] = h
    a_s = jnp.sum(h * as_ref[...], axis=1, keepdims=True)
    a_d = jnp.sum(h * ad_ref[...], axis=1, keepdims=True)
    asrc_ref[...] = a_s
    adst_ref[...] = a_d
    bm = jnp.max(a_s)

    bm2 = bm.reshape(1, 1)

    @pl.when(i == 0)
    def _():
        gmax_ref[...] = bm2

    @pl.when(i > 0)
    def _():
        gmax_ref[...] = jnp.maximum(gmax_ref[...], bm2)


def _tc_mid_body(part_ref, den_ref, b_ref, w_ref, as_ref, ad_ref,
                 h_ref, asrc_ref, adst_ref, gmax_ref):
    i = pl.program_id(0)
    agg = part_ref[0] + part_ref[1]
    den = den_ref[:, 0:1] + den_ref[:, 1:2]
    out = agg / (den + 1e-16) + b_ref[...]
    hx = jnp.maximum(out, 0.0)
    h2 = jnp.dot(hx, w_ref[...], preferred_element_type=jnp.float32)
    h_ref[...] = h2
    a_s = jnp.sum(h2 * as_ref[...], axis=1, keepdims=True)
    a_d = jnp.sum(h2 * ad_ref[...], axis=1, keepdims=True)
    asrc_ref[...] = a_s
    adst_ref[...] = a_d
    bm = jnp.max(a_s)

    bm2 = bm.reshape(1, 1)

    @pl.when(i == 0)
    def _():
        gmax_ref[...] = bm2

    @pl.when(i > 0)
    def _():
        gmax_ref[...] = jnp.maximum(gmax_ref[...], bm2)


def _tc3_body(part_ref, den_ref, b_ref, wa_ref, wb_ref, mb_ref,
              p_ref, q_ref):
    agg = part_ref[0] + part_ref[1]
    den = den_ref[:, 0:1] + den_ref[:, 1:2]
    hf = agg / (den + 1e-16) + b_ref[...]
    p_ref[...] = (jnp.dot(hf, wa_ref[...], preferred_element_type=jnp.float32)
                  + mb_ref[...])
    q_ref[...] = jnp.dot(hf, wb_ref[...], preferred_element_type=jnp.float32)


_EBLK = 2000
_NEBLK = E // _EBLK
_LOG_PRIOR = float(np.log(np.float32(1.0 / 3.0) + np.float32(1e-12)))


def _tc4_body(s_ref, w_ref, b_ref,
              logits_ref, probs_ref, kl_ref, rec_ref):
    i = pl.program_id(0)
    hid = jnp.maximum(s_ref[...], 0.0)
    lf = jnp.dot(hid, w_ref[...], preferred_element_type=jnp.float32) + b_ref[...]
    col = lax.broadcasted_iota(jnp.int32, lf.shape, 1)
    valid = (col % H) < 3
    lfm = jnp.where(valid, lf, -jnp.inf)
    a, bb = lfm[:, :H], lfm[:, H:]
    m = jnp.concatenate(
        [jnp.broadcast_to(jnp.max(a, axis=1, keepdims=True), a.shape),
         jnp.broadcast_to(jnp.max(bb, axis=1, keepdims=True), bb.shape)],
        axis=1)
    e = jnp.where(valid, jnp.exp(lf - m), 0.0)
    ea, eb = e[:, :H], e[:, H:]
    den = jnp.concatenate(
        [jnp.broadcast_to(jnp.sum(ea, axis=1, keepdims=True), ea.shape),
         jnp.broadcast_to(jnp.sum(eb, axis=1, keepdims=True), eb.shape)],
        axis=1)
    p = e / den
    logits_ref[...] = jnp.concatenate([lf[:, 0:3], lf[:, H:H + 3]], axis=1)
    probs_ref[...] = jnp.concatenate([p[:, 0:3], p[:, H:H + 3]], axis=1)
    lp = jnp.log(jnp.where(valid, p, 1.0))
    kl = jnp.where(valid, p * (lp - _LOG_PRIOR), 0.0)
    e02 = jnp.where(col % H == 1, 0.0, e)
    p02a = jnp.sum(e02[:, :H], axis=1, keepdims=True) / den[:, 0:1]
    p02b = jnp.sum(e02[:, H:], axis=1, keepdims=True) / den[:, H:H + 1]
    rec = jnp.log(p02a + 1e-12) + jnp.log(p02b + 1e-12)
    kl_s = jnp.sum(kl)
    rec_s = jnp.sum(rec)

    kl_s2 = kl_s.reshape(1, 1)
    rec_s2 = rec_s.reshape(1, 1)

    @pl.when(i == 0)
    def _():
        kl_ref[...] = kl_s2
        rec_ref[...] = rec_s2

    @pl.when(i > 0)
    def _():
        kl_ref[...] = kl_ref[...] + kl_s2
        rec_ref[...] = rec_ref[...] + rec_s2


def _node_stage1(x, W1, att_src1, att_dst1):
    return pl.pallas_call(
        _tc1_body,
        grid=(_NBLK,),
        in_specs=[
            pl.BlockSpec((_BLK, D), lambda i: (i, 0)),
            pl.BlockSpec((D, H), lambda i: (0, 0)),
            pl.BlockSpec((1, H), lambda i: (0, 0)),
            pl.BlockSpec((1, H), lambda i: (0, 0)),
        ],
        out_specs=[
            pl.BlockSpec((_BLK, H), lambda i: (i, 0)),
            pl.BlockSpec((_BLK, 1), lambda i: (i, 0)),
            pl.BlockSpec((_BLK, 1), lambda i: (i, 0)),
            pl.BlockSpec((1, 1), lambda i: (0, 0)),
        ],
        out_shape=[
            jax.ShapeDtypeStruct((N, H), jnp.float32),
            jax.ShapeDtypeStruct((N, 1), jnp.float32),
            jax.ShapeDtypeStruct((N, 1), jnp.float32),
            jax.ShapeDtypeStruct((1, 1), jnp.float32),
        ],
    )(x, W1, att_src1.reshape(1, H), att_dst1.reshape(1, H))


def _node_stage2(part, den_t, bias1, W2, att_src2, att_dst2):
    return pl.pallas_call(
        _tc_mid_body,
        grid=(_NBLK,),
        in_specs=[
            pl.BlockSpec((NC, _BLK, H), lambda i: (0, i, 0)),
            pl.BlockSpec((_BLK, NC), lambda i: (i, 0)),
            pl.BlockSpec((1, H), lambda i: (0, 0)),
            pl.BlockSpec((H, H), lambda i: (0, 0)),
            pl.BlockSpec((1, H), lambda i: (0, 0)),
            pl.BlockSpec((1, H), lambda i: (0, 0)),
        ],
        out_specs=[
            pl.BlockSpec((_BLK, H), lambda i: (i, 0)),
            pl.BlockSpec((_BLK, 1), lambda i: (i, 0)),
            pl.BlockSpec((_BLK, 1), lambda i: (i, 0)),
            pl.BlockSpec((1, 1), lambda i: (0, 0)),
        ],
        out_shape=[
            jax.ShapeDtypeStruct((N, H), jnp.float32),
            jax.ShapeDtypeStruct((N, 1), jnp.float32),
            jax.ShapeDtypeStruct((N, 1), jnp.float32),
            jax.ShapeDtypeStruct((1, 1), jnp.float32),
        ],
    )(part, den_t, bias1.reshape(1, H), W2,
      att_src2.reshape(1, H), att_dst2.reshape(1, H))


def _node_stage3(part, den_t, bias2, mW1, mb1):
    return pl.pallas_call(
        _tc3_body,
        grid=(_NBLK,),
        in_specs=[
            pl.BlockSpec((NC, _BLK, H), lambda i: (0, i, 0)),
            pl.BlockSpec((_BLK, NC), lambda i: (i, 0)),
            pl.BlockSpec((1, H), lambda i: (0, 0)),
            pl.BlockSpec((H, H), lambda i: (0, 0)),
            pl.BlockSpec((H, H), lambda i: (0, 0)),
            pl.BlockSpec((1, H), lambda i: (0, 0)),
        ],
        out_specs=[
            pl.BlockSpec((_BLK, H), lambda i: (i, 0)),
            pl.BlockSpec((_BLK, H), lambda i: (i, 0)),
        ],
        out_shape=[
            jax.ShapeDtypeStruct((N, H), jnp.float32),
            jax.ShapeDtypeStruct((N, H), jnp.float32),
        ],
    )(part, den_t, bias2.reshape(1, H), mW1[:H], mW1[H:], mb1.reshape(1, H))


def _edge_stage(S2, mW2, mb2):
    w_pad = (jnp.zeros((2 * H, 128), jnp.float32)
             .at[:H, :3].set(mW2).at[H:, H:H + 3].set(mW2))
    b_pad = (jnp.zeros((1, 128), jnp.float32)
             .at[0, :3].set(mb2).at[0, H:H + 3].set(mb2))
    return pl.pallas_call(
        _tc4_body,
        grid=(_NEBLK,),
        in_specs=[
            pl.BlockSpec((_EBLK // 2, 2 * H), lambda i: (i, 0)),
            pl.BlockSpec((2 * H, 128), lambda i: (0, 0)),
            pl.BlockSpec((1, 128), lambda i: (0, 0)),
        ],
        out_specs=[
            pl.BlockSpec((_EBLK // 2, 6), lambda i: (i, 0)),
            pl.BlockSpec((_EBLK // 2, 6), lambda i: (i, 0)),
            pl.BlockSpec((1, 1), lambda i: (0, 0)),
            pl.BlockSpec((1, 1), lambda i: (0, 0)),
        ],
        out_shape=[
            jax.ShapeDtypeStruct((E // 2, 6), jnp.float32),
            jax.ShapeDtypeStruct((E // 2, 6), jnp.float32),
            jax.ShapeDtypeStruct((1, 1), jnp.float32),
            jax.ShapeDtypeStruct((1, 1), jnp.float32),
        ],
    )(S2, w_pad, b_pad)


def kernel(x, edge_index, W1, att_src1, att_dst1, bias1,
           W2, att_src2, att_dst2, bias2, mW1, mb1, mW2, mb2):
    src = edge_index[0]
    dst = edge_index[1]
    loop_idx = jnp.arange(N, dtype=jnp.int32)

    n_dummy = E_IDX - EP
    src_full = jnp.concatenate(
        [src, loop_idx, jnp.arange(n_dummy, dtype=jnp.int32) % N])
    dst_full = jnp.concatenate(
        [dst, loop_idx, jnp.zeros((n_dummy,), jnp.int32)])

    n_d2 = E2_IDX - E
    d2 = jnp.arange(n_d2, dtype=jnp.int32) % N
    row_full = jnp.concatenate([src, d2])
    col_full = jnp.concatenate([dst, d2])

    h1, as1, ad1, gm1 = _node_stage1(x, W1, att_src1, att_dst1)
    gvec1 = jnp.broadcast_to(gm1.reshape(()), (L,))
    part1, den1 = _sc_conv_kernel()(h1, as1.reshape(N), ad1.reshape(N), gvec1,
                                    src_full, dst_full)

    h2, as2, ad2, gm2 = _node_stage2(part1, den1.T, bias1, W2,
                                     att_src2, att_dst2)
    gvec2 = jnp.broadcast_to(gm2.reshape(()), (L,))
    part2, den2 = _sc_conv_kernel()(h2, as2.reshape(N), ad2.reshape(N), gvec2,
                                    src_full, dst_full)

    P, Q = _node_stage3(part2, den2.T, bias2, mW1, mb1)
    S2 = _sc_pair_kernel()(P, Q, row_full, col_full)
    logits2, probs2, kl_sum, rec_sum = _edge_stage(S2, mW2, mb2)
    logits = logits2.reshape(E, 3)
    probs = probs2.reshape(E, 3)

    struct_loss = (kl_sum.reshape(()) - rec_sum.reshape(())) / jnp.float32(E)
    return (logits, probs, struct_loss)


# speedup vs baseline: 1.1330x; 1.1330x over previous
import functools

import jax
import jax.numpy as jnp
import numpy as np
from jax import lax
from jax.experimental import pallas as pl
from jax.experimental.pallas import tpu as pltpu
from jax.experimental.pallas import tpu_sc as plsc

N, E, D, H = 10000, 320000, 128, 64
NC, NS, L = 2, 16, 16
NW = NC * NS
CH = 128

EP = E + N
NCHUNK = 82
PER_W = NCHUNK * CH
E_PAD = NW * PER_W
E_IDX = E_PAD + 2 * CH

NCHUNK2 = 80
PER_W2 = NCHUNK2 * CH
E2_PAD = NW * PER_W2
E2_IDX = E2_PAD + 2 * CH

N_ACC = 10240
ROWS_PER_SUB = N_ACC // NS

@functools.cache
def _sc_mesh():
    return plsc.VectorSubcoreMesh(core_axis_name="c", subcore_axis_name="s",
                                  num_cores=NC, num_subcores=NS)


def _sc_conv_body(h_hbm, asrc_hbm, adst_hbm, gmax_hbm, src_hbm, dst_hbm,
                  out_hbm, den_hbm,
                  asrc_v, adst_v, gmax_v,
                  src_v0, dst_v0, rows_v0, src_v1, dst_v1, rows_v1, p_v,
                  zbuf, zden, acc_sh, den_sh,
                  gsem0, gsem1, si0, di0, si1, di1):
    cid = lax.axis_index("c")
    sid = lax.axis_index("s")
    wid = sid * NC + cid

    pltpu.sync_copy(asrc_hbm, asrc_v)
    pltpu.sync_copy(adst_hbm, adst_v)
    pltpu.sync_copy(gmax_hbm, gmax_v)

    def _zrow(i, _):
        for q in range(4):
            zbuf[i, pl.ds(q * L, L)] = jnp.zeros((L,), jnp.float32)
        return 0
    lax.fori_loop(0, CH, _zrow, 0)

    def _zden(i, _):
        zden[pl.ds(i * L, L)] = jnp.zeros((L,), jnp.float32)
        return 0
    lax.fori_loop(0, CH // L, _zden, 0)

    for t in range(ROWS_PER_SUB // CH):
        pltpu.sync_copy(zbuf, acc_sh.at[pl.ds(sid * ROWS_PER_SUB + t * CH, CH)])
        pltpu.sync_copy(zden, den_sh.at[pl.ds(sid * ROWS_PER_SUB + t * CH, CH)])
    plsc.subcore_barrier()

    base = wid * PER_W
    bufs = ((src_v0, dst_v0, rows_v0, gsem0, si0, di0),
            (src_v1, dst_v1, rows_v1, gsem1, si1, di1))

    def _process(k, cur, nxt):
        src_c, dst_c, rows_c, gsem_c, _, _ = cur
        src_n, dst_n, rows_n, gsem_n, si_n, di_n = nxt
        pltpu.make_async_copy(src_hbm.at[pl.ds(0, CH)], src_n, si_n).wait()
        pltpu.make_async_copy(dst_hbm.at[pl.ds(0, CH)], dst_n, di_n).wait()
        pltpu.async_copy(h_hbm.at[src_n], rows_n, gsem_n)
        cb = base + k * CH
        gvec = gmax_v[...]
        for g in range(CH // L):
            s_idx = src_c[pl.ds(g * L, L)]
            d_idx = dst_c[pl.ds(g * L, L)]
            a_s = plsc.load_gather(asrc_v, [s_idx])
            a_d = plsc.load_gather(adst_v, [d_idx])
            al = a_s + a_d
            al = jnp.where(al >= 0.0, al, 0.2 * al)
            m = gvec + a_d
            m = jnp.where(m >= 0.0, m, 0.2 * m)
            p = jnp.exp(al - m)
            pos = cb + g * L + lax.iota(jnp.int32, L)
            p = jnp.where(pos < EP, p, 0.0)
            p_v[pl.ds(g * L, L)] = p
        pltpu.make_async_copy(src_hbm.at[pl.ds(0, CH)], rows_c, gsem_c).wait()

        def _scale(j4, _):
            j = j4 * 4
            for u in range(4):
                pj = plsc.load_gather(p_v, [jnp.full((L,), j + u, jnp.int32)])
                for q in range(4):
                    rows_c[j + u, pl.ds(q * L, L)] = (
                        rows_c[j + u, pl.ds(q * L, L)] * pj)
            return 0
        lax.fori_loop(0, CH // 4, _scale, 0)
        pltpu.sync_copy(rows_c, acc_sh.at[dst_c], add=True)
        pltpu.sync_copy(p_v, den_sh.at[dst_c], add=True)
        nb = base + (k + 2) * CH
        pltpu.async_copy(src_hbm.at[pl.ds(nb, CH)], src_c, cur[4])
        pltpu.async_copy(dst_hbm.at[pl.ds(nb, CH)], dst_c, cur[5])

    pltpu.sync_copy(src_hbm.at[pl.ds(base, CH)], src_v0)
    pltpu.sync_copy(dst_hbm.at[pl.ds(base, CH)], dst_v0)
    pltpu.async_copy(src_hbm.at[pl.ds(base + CH, CH)], src_v1, si1)
    pltpu.async_copy(dst_hbm.at[pl.ds(base + CH, CH)], dst_v1, di1)
    pltpu.async_copy(h_hbm.at[src_v0], rows_v0, gsem0)

    def _pair_steps(t, _):
        _process(2 * t, bufs[0], bufs[1])
        _process(2 * t + 1, bufs[1], bufs[0])
        return 0
    lax.fori_loop(0, NCHUNK // 2, _pair_steps, 0)

    pltpu.make_async_copy(src_hbm.at[pl.ds(0, CH)], rows_v0, gsem0).wait()
    pltpu.make_async_copy(src_hbm.at[pl.ds(0, CH)], src_v1, si1).wait()
    pltpu.make_async_copy(dst_hbm.at[pl.ds(0, CH)], dst_v1, di1).wait()
    plsc.subcore_barrier()

    for t in range(ROWS_PER_SUB // CH):
        o = sid * ROWS_PER_SUB + t * CH
        pltpu.sync_copy(acc_sh.at[pl.ds(o, CH)], out_hbm.at[cid, pl.ds(o, CH)])
        pltpu.sync_copy(den_sh.at[pl.ds(o, CH)], den_hbm.at[cid, pl.ds(o, CH)])


@functools.cache
def _sc_conv_kernel():
  return pl.kernel(
    _sc_conv_body,
    out_type=(jax.ShapeDtypeStruct((NC, N_ACC, H), jnp.float32),
              jax.ShapeDtypeStruct((NC, N_ACC), jnp.float32)),
    mesh=_sc_mesh(),
    compiler_params=pltpu.CompilerParams(needs_layout_passes=False, use_tc_tiling_on_sc=False),
    scratch_types=[
        pltpu.VMEM((N,), jnp.float32),
        pltpu.VMEM((N,), jnp.float32),
        pltpu.VMEM((L,), jnp.float32),
        pltpu.VMEM((CH,), jnp.int32),
        pltpu.VMEM((CH,), jnp.int32),
        pltpu.VMEM((CH, H), jnp.float32),
        pltpu.VMEM((CH,), jnp.int32),
        pltpu.VMEM((CH,), jnp.int32),
        pltpu.VMEM((CH, H), jnp.float32),
        pltpu.VMEM((CH,), jnp.float32),
        pltpu.VMEM((CH, H), jnp.float32),
        pltpu.VMEM((CH,), jnp.float32),
        pltpu.VMEM_SHARED((N_ACC, H), jnp.float32),
        pltpu.VMEM_SHARED((N_ACC,), jnp.float32),
        pltpu.SemaphoreType.DMA,
        pltpu.SemaphoreType.DMA,
        pltpu.SemaphoreType.DMA,
        pltpu.SemaphoreType.DMA,
        pltpu.SemaphoreType.DMA,
        pltpu.SemaphoreType.DMA,
    ],
  )


def _sc_pair_body(p_hbm, q_hbm, row_hbm, col_hbm, s_hbm,
                  row_v0, col_v0, pbuf0, qbuf0, sbuf0,
                  row_v1, col_v1, pbuf1, qbuf1, sbuf1,
                  gp0, gq0, gp1, gq1, ri0, ci0, ri1, ci1, wsem0, wsem1):
    cid = lax.axis_index("c")
    sid = lax.axis_index("s")
    wid = sid * NC + cid
    base = wid * PER_W2
    bufs = ((row_v0, col_v0, pbuf0, qbuf0, sbuf0, gp0, gq0, ri0, ci0, wsem0),
            (row_v1, col_v1, pbuf1, qbuf1, sbuf1, gp1, gq1, ri1, ci1, wsem1))

    def _process(k, cur, nxt):
        row_c, col_c, pb_c, qb_c, sb_c, gp_c, gq_c, ri_c, ci_c, ws_c = cur
        row_n, col_n, pb_n, qb_n, sb_n, gp_n, gq_n, ri_n, ci_n, ws_n = nxt
        pltpu.make_async_copy(row_hbm.at[pl.ds(0, CH)], row_n, ri_n).wait()
        pltpu.make_async_copy(col_hbm.at[pl.ds(0, CH)], col_n, ci_n).wait()
        pltpu.async_copy(p_hbm.at[row_n], pb_n, gp_n)
        pltpu.async_copy(q_hbm.at[col_n], qb_n, gq_n)
        pltpu.make_async_copy(row_hbm.at[pl.ds(0, CH)], pb_c, gp_c).wait()
        pltpu.make_async_copy(row_hbm.at[pl.ds(0, CH)], qb_c, gq_c).wait()

        @pl.when(k >= 2)
        def _():
            pltpu.make_async_copy(row_hbm.at[pl.ds(0, CH)], sb_c, ws_c).wait()

        def _add(i, _):
            for q in range(4):
                sb_c[i, pl.ds(q * L, L)] = (pb_c[i, pl.ds(q * L, L)]
                                            + qb_c[i, pl.ds(q * L, L)])
            return 0
        lax.fori_loop(0, CH, _add, 0)
        cb = base + k * CH
        pltpu.async_copy(sb_c, s_hbm.at[pl.ds(cb, CH)], ws_c)
        nb = base + (k + 2) * CH
        pltpu.async_copy(row_hbm.at[pl.ds(nb, CH)], row_c, ri_c)
        pltpu.async_copy(col_hbm.at[pl.ds(nb, CH)], col_c, ci_c)

    pltpu.sync_copy(row_hbm.at[pl.ds(base, CH)], row_v0)
    pltpu.sync_copy(col_hbm.at[pl.ds(base, CH)], col_v0)
    pltpu.async_copy(row_hbm.at[pl.ds(base + CH, CH)], row_v1, ri1)
    pltpu.async_copy(col_hbm.at[pl.ds(base + CH, CH)], col_v1, ci1)
    pltpu.async_copy(p_hbm.at[row_v0], pbuf0, gp0)
    pltpu.async_copy(q_hbm.at[col_v0], qbuf0, gq0)

    def _pair_steps(t, _):
        _process(2 * t, bufs[0], bufs[1])
        _process(2 * t + 1, bufs[1], bufs[0])
        return 0
    lax.fori_loop(0, NCHUNK2 // 2, _pair_steps, 0)

    pltpu.make_async_copy(row_hbm.at[pl.ds(0, CH)], pbuf0, gp0).wait()
    pltpu.make_async_copy(row_hbm.at[pl.ds(0, CH)], qbuf0, gq0).wait()
    pltpu.make_async_copy(row_hbm.at[pl.ds(0, CH)], row_v1, ri1).wait()
    pltpu.make_async_copy(col_hbm.at[pl.ds(0, CH)], col_v1, ci1).wait()
    pltpu.make_async_copy(row_hbm.at[pl.ds(0, CH)], sbuf0, wsem0).wait()
    pltpu.make_async_copy(row_hbm.at[pl.ds(0, CH)], sbuf1, wsem1).wait()


@functools.cache
def _sc_pair_kernel():
  return pl.kernel(
    _sc_pair_body,
    out_type=jax.ShapeDtypeStruct((E2_PAD, 2 * H), jnp.float32),
    mesh=_sc_mesh(),
    compiler_params=pltpu.CompilerParams(needs_layout_passes=False, use_tc_tiling_on_sc=False),
    scratch_types=(
        [pltpu.VMEM((CH,), jnp.int32), pltpu.VMEM((CH,), jnp.int32),
         pltpu.VMEM((CH, H), jnp.float32), pltpu.VMEM((CH, H), jnp.float32),
         pltpu.VMEM((CH, 2 * H), jnp.float32)] * 2
        + [pltpu.SemaphoreType.DMA] * 10
    ),
  )


_BLK = 1000
_NBLK = N // _BLK


def _tc1_body(x_ref, w_ref, as_ref, ad_ref,
              h_ref, asrc_ref, adst_ref, gmax_ref):
    i = pl.program_id(0)
    h = jnp.dot(x_ref[...], w_ref[...], preferred_element_type=jnp.float32)
    h_ref[...] = h
    a_s = jnp.sum(h * as_ref[...], axis=1, keepdims=True)
    a_d = jnp.sum(h * ad_ref[...], axis=1, keepdims=True)
    asrc_ref[...] = a_s
    adst_ref[...] = a_d
    bm = jnp.max(a_s)

    bm2 = bm.reshape(1, 1)

    @pl.when(i == 0)
    def _():
        gmax_ref[...] = bm2

    @pl.when(i > 0)
    def _():
        gmax_ref[...] = jnp.maximum(gmax_ref[...], bm2)


def _tc_mid_body(part_ref, den_ref, b_ref, w_ref, as_ref, ad_ref,
                 h_ref, asrc_ref, adst_ref, gmax_ref):
    i = pl.program_id(0)
    agg = part_ref[0] + part_ref[1]
    den = den_ref[:, 0:1] + den_ref[:, 1:2]
    out = agg / (den + 1e-16) + b_ref[...]
    hx = jnp.maximum(out, 0.0)
    h2 = jnp.dot(hx, w_ref[...], preferred_element_type=jnp.float32)
    h_ref[...] = h2
    a_s = jnp.sum(h2 * as_ref[...], axis=1, keepdims=True)
    a_d = jnp.sum(h2 * ad_ref[...], axis=1, keepdims=True)
    asrc_ref[...] = a_s
    adst_ref[...] = a_d
    bm = jnp.max(a_s)

    bm2 = bm.reshape(1, 1)

    @pl.when(i == 0)
    def _():
        gmax_ref[...] = bm2

    @pl.when(i > 0)
    def _():
        gmax_ref[...] = jnp.maximum(gmax_ref[...], bm2)


def _tc3_body(part_ref, den_ref, b_ref, wa_ref, wb_ref, mb_ref,
              p_ref, q_ref):
    agg = part_ref[0] + part_ref[1]
    den = den_ref[:, 0:1] + den_ref[:, 1:2]
    hf = agg / (den + 1e-16) + b_ref[...]
    p_ref[...] = (jnp.dot(hf, wa_ref[...], preferred_element_type=jnp.float32)
                  + mb_ref[...])
    q_ref[...] = jnp.dot(hf, wb_ref[...], preferred_element_type=jnp.float32)


_EBLK = 2000
_NEBLK = E // _EBLK
_LOG_PRIOR = float(np.log(np.float32(1.0 / 3.0) + np.float32(1e-12)))


def _tc4_body(s_ref, w_ref, b_ref,
              logits_ref, probs_ref, kl_ref, rec_ref):
    i = pl.program_id(0)
    hid = jnp.maximum(s_ref[:, :H], 0.0)
    lf = jnp.dot(hid, w_ref[...], preferred_element_type=jnp.float32) + b_ref[...]
    col = lax.broadcasted_iota(jnp.int32, lf.shape, 1)
    valid = col < 3
    lfm = jnp.where(valid, lf, -jnp.inf)
    m = jnp.max(lfm, axis=1, keepdims=True)
    e = jnp.where(valid, jnp.exp(lf - m), 0.0)
    den = jnp.sum(e, axis=1, keepdims=True)
    p = e / den
    logits_ref[...] = lf[:, 0:3]
    probs_ref[...] = p[:, 0:3]
    lp = jnp.log(jnp.where(valid, p, 1.0))
    kl = jnp.where(valid, p * (lp - _LOG_PRIOR), 0.0)
    p02 = jnp.sum(jnp.where(col == 1, 0.0, e), axis=1, keepdims=True) / den
    rec = jnp.log(p02 + 1e-12)
    kl_s = jnp.sum(kl)
    rec_s = jnp.sum(rec)

    kl_s2 = kl_s.reshape(1, 1)
    rec_s2 = rec_s.reshape(1, 1)

    @pl.when(i == 0)
    def _():
        kl_ref[...] = kl_s2
        rec_ref[...] = rec_s2

    @pl.when(i > 0)
    def _():
        kl_ref[...] = kl_ref[...] + kl_s2
        rec_ref[...] = rec_ref[...] + rec_s2


def _node_stage1(x, W1, att_src1, att_dst1):
    return pl.pallas_call(
        _tc1_body,
        grid=(_NBLK,),
        in_specs=[
            pl.BlockSpec((_BLK, D), lambda i: (i, 0)),
            pl.BlockSpec((D, H), lambda i: (0, 0)),
            pl.BlockSpec((1, H), lambda i: (0, 0)),
            pl.BlockSpec((1, H), lambda i: (0, 0)),
        ],
        out_specs=[
            pl.BlockSpec((_BLK, H), lambda i: (i, 0)),
            pl.BlockSpec((_BLK, 1), lambda i: (i, 0)),
            pl.BlockSpec((_BLK, 1), lambda i: (i, 0)),
            pl.BlockSpec((1, 1), lambda i: (0, 0)),
        ],
        out_shape=[
            jax.ShapeDtypeStruct((N, H), jnp.float32),
            jax.ShapeDtypeStruct((N, 1), jnp.float32),
            jax.ShapeDtypeStruct((N, 1), jnp.float32),
            jax.ShapeDtypeStruct((1, 1), jnp.float32),
        ],
    )(x, W1, att_src1.reshape(1, H), att_dst1.reshape(1, H))


def _node_stage2(part, den_t, bias1, W2, att_src2, att_dst2):
    return pl.pallas_call(
        _tc_mid_body,
        grid=(_NBLK,),
        in_specs=[
            pl.BlockSpec((NC, _BLK, H), lambda i: (0, i, 0)),
            pl.BlockSpec((_BLK, NC), lambda i: (i, 0)),
            pl.BlockSpec((1, H), lambda i: (0, 0)),
            pl.BlockSpec((H, H), lambda i: (0, 0)),
            pl.BlockSpec((1, H), lambda i: (0, 0)),
            pl.BlockSpec((1, H), lambda i: (0, 0)),
        ],
        out_specs=[
            pl.BlockSpec((_BLK, H), lambda i: (i, 0)),
            pl.BlockSpec((_BLK, 1), lambda i: (i, 0)),
            pl.BlockSpec((_BLK, 1), lambda i: (i, 0)),
            pl.BlockSpec((1, 1), lambda i: (0, 0)),
        ],
        out_shape=[
            jax.ShapeDtypeStruct((N, H), jnp.float32),
            jax.ShapeDtypeStruct((N, 1), jnp.float32),
            jax.ShapeDtypeStruct((N, 1), jnp.float32),
            jax.ShapeDtypeStruct((1, 1), jnp.float32),
        ],
    )(part, den_t, bias1.reshape(1, H), W2,
      att_src2.reshape(1, H), att_dst2.reshape(1, H))


def _node_stage3(part, den_t, bias2, mW1, mb1):
    return pl.pallas_call(
        _tc3_body,
        grid=(_NBLK,),
        in_specs=[
            pl.BlockSpec((NC, _BLK, H), lambda i: (0, i, 0)),
            pl.BlockSpec((_BLK, NC), lambda i: (i, 0)),
            pl.BlockSpec((1, H), lambda i: (0, 0)),
            pl.BlockSpec((H, H), lambda i: (0, 0)),
            pl.BlockSpec((H, H), lambda i: (0, 0)),
            pl.BlockSpec((1, H), lambda i: (0, 0)),
        ],
        out_specs=[
            pl.BlockSpec((_BLK, H), lambda i: (i, 0)),
            pl.BlockSpec((_BLK, H), lambda i: (i, 0)),
        ],
        out_shape=[
            jax.ShapeDtypeStruct((N, H), jnp.float32),
            jax.ShapeDtypeStruct((N, H), jnp.float32),
        ],
    )(part, den_t, bias2.reshape(1, H), mW1[:H], mW1[H:], mb1.reshape(1, H))


def _edge_stage(S2, mW2, mb2):
    w_pad = jnp.zeros((H, 128), jnp.float32).at[:, :3].set(mW2)
    b_pad = jnp.zeros((1, 128), jnp.float32).at[0, :3].set(mb2)
    return pl.pallas_call(
        _tc4_body,
        grid=(_NEBLK,),
        in_specs=[
            pl.BlockSpec((_EBLK, 2 * H), lambda i: (i, 0)),
            pl.BlockSpec((H, 128), lambda i: (0, 0)),
            pl.BlockSpec((1, 128), lambda i: (0, 0)),
        ],
        out_specs=[
            pl.BlockSpec((_EBLK, 3), lambda i: (i, 0)),
            pl.BlockSpec((_EBLK, 3), lambda i: (i, 0)),
            pl.BlockSpec((1, 1), lambda i: (0, 0)),
            pl.BlockSpec((1, 1), lambda i: (0, 0)),
        ],
        out_shape=[
            jax.ShapeDtypeStruct((E, 3), jnp.float32),
            jax.ShapeDtypeStruct((E, 3), jnp.float32),
            jax.ShapeDtypeStruct((1, 1), jnp.float32),
            jax.ShapeDtypeStruct((1, 1), jnp.float32),
        ],
    )(S2, w_pad, b_pad)


def kernel(x, edge_index, W1, att_src1, att_dst1, bias1,
           W2, att_src2, att_dst2, bias2, mW1, mb1, mW2, mb2):
    src = edge_index[0]
    dst = edge_index[1]
    loop_idx = jnp.arange(N, dtype=jnp.int32)

    n_dummy = E_IDX - EP
    src_full = jnp.concatenate(
        [src, loop_idx, jnp.arange(n_dummy, dtype=jnp.int32) % N])
    dst_full = jnp.concatenate(
        [dst, loop_idx, jnp.zeros((n_dummy,), jnp.int32)])

    n_d2 = E2_IDX - E
    d2 = jnp.arange(n_d2, dtype=jnp.int32) % N
    row_full = jnp.concatenate([src, d2])
    col_full = jnp.concatenate([dst, d2])

    h1, as1, ad1, gm1 = _node_stage1(x, W1, att_src1, att_dst1)
    gvec1 = jnp.broadcast_to(gm1.reshape(()), (L,))
    part1, den1 = _sc_conv_kernel()(h1, as1.reshape(N), ad1.reshape(N), gvec1,
                                    src_full, dst_full)

    h2, as2, ad2, gm2 = _node_stage2(part1, den1.T, bias1, W2,
                                     att_src2, att_dst2)
    gvec2 = jnp.broadcast_to(gm2.reshape(()), (L,))
    part2, den2 = _sc_conv_kernel()(h2, as2.reshape(N), ad2.reshape(N), gvec2,
                                    src_full, dst_full)

    P, Q = _node_stage3(part2, den2.T, bias2, mW1, mb1)
    S2 = _sc_pair_kernel()(P, Q, row_full, col_full)
    logits, probs, kl_sum, rec_sum = _edge_stage(S2, mW2, mb2)

    struct_loss = (kl_sum.reshape(()) - rec_sum.reshape(())) / jnp.float32(E)
    return (logits, probs, struct_loss)


# speedup vs baseline: 1.2281x; 1.0839x over previous
import functools

import jax
import jax.numpy as jnp
import numpy as np
from jax import lax
from jax.experimental import pallas as pl
from jax.experimental.pallas import tpu as pltpu
from jax.experimental.pallas import tpu_sc as plsc

N, E, D, H = 10000, 320000, 128, 64
NC, NS, L = 2, 16, 16
NW = NC * NS
CH = 128

EP = E + N
NCHUNK = 82
PER_W = NCHUNK * CH
E_PAD = NW * PER_W
E_IDX = E_PAD + 2 * CH

EH = E // 2
NCHUNK2 = 40
PER_W2 = NCHUNK2 * CH
E2_PAD = NW * PER_W2
E2_IDX = E2_PAD + 2 * CH

N_ACC = 10240
ROWS_PER_SUB = N_ACC // NS

@functools.cache
def _sc_mesh():
    return plsc.VectorSubcoreMesh(core_axis_name="c", subcore_axis_name="s",
                                  num_cores=NC, num_subcores=NS)


def _sc_conv_body(h_hbm, asrc_hbm, adst_hbm, gmax_hbm, src_hbm, dst_hbm,
                  out_hbm, den_hbm,
                  asrc_v, adst_v, gmax_v,
                  src_v0, dst_v0, rows_v0, src_v1, dst_v1, rows_v1, p_v,
                  zbuf, zden, acc_sh, den_sh,
                  gsem0, gsem1, si0, di0, si1, di1):
    cid = lax.axis_index("c")
    sid = lax.axis_index("s")
    wid = sid * NC + cid

    pltpu.sync_copy(asrc_hbm, asrc_v)
    pltpu.sync_copy(adst_hbm, adst_v)
    pltpu.sync_copy(gmax_hbm, gmax_v)

    def _zrow(i, _):
        for q in range(4):
            zbuf[i, pl.ds(q * L, L)] = jnp.zeros((L,), jnp.float32)
        return 0
    lax.fori_loop(0, CH, _zrow, 0)

    def _zden(i, _):
        zden[pl.ds(i * L, L)] = jnp.zeros((L,), jnp.float32)
        return 0
    lax.fori_loop(0, CH // L, _zden, 0)

    for t in range(ROWS_PER_SUB // CH):
        pltpu.sync_copy(zbuf, acc_sh.at[pl.ds(sid * ROWS_PER_SUB + t * CH, CH)])
        pltpu.sync_copy(zden, den_sh.at[pl.ds(sid * ROWS_PER_SUB + t * CH, CH)])
    plsc.subcore_barrier()

    base = wid * PER_W
    bufs = ((src_v0, dst_v0, rows_v0, gsem0, si0, di0),
            (src_v1, dst_v1, rows_v1, gsem1, si1, di1))

    def _process(k, cur, nxt):
        src_c, dst_c, rows_c, gsem_c, _, _ = cur
        src_n, dst_n, rows_n, gsem_n, si_n, di_n = nxt
        pltpu.make_async_copy(src_hbm.at[pl.ds(0, CH)], src_n, si_n).wait()
        pltpu.make_async_copy(dst_hbm.at[pl.ds(0, CH)], dst_n, di_n).wait()
        pltpu.async_copy(h_hbm.at[src_n], rows_n, gsem_n)
        cb = base + k * CH
        gvec = gmax_v[...]
        for g in range(CH // L):
            s_idx = src_c[pl.ds(g * L, L)]
            d_idx = dst_c[pl.ds(g * L, L)]
            a_s = plsc.load_gather(asrc_v, [s_idx])
            a_d = plsc.load_gather(adst_v, [d_idx])
            al = a_s + a_d
            al = jnp.where(al >= 0.0, al, 0.2 * al)
            m = gvec + a_d
            m = jnp.where(m >= 0.0, m, 0.2 * m)
            p = jnp.exp(al - m)
            pos = cb + g * L + lax.iota(jnp.int32, L)
            p = jnp.where(pos < EP, p, 0.0)
            p_v[pl.ds(g * L, L)] = p
        pltpu.make_async_copy(src_hbm.at[pl.ds(0, CH)], rows_c, gsem_c).wait()

        def _scale(j4, _):
            j = j4 * 4
            for u in range(4):
                pj = plsc.load_gather(p_v, [jnp.full((L,), j + u, jnp.int32)])
                for q in range(4):
                    rows_c[j + u, pl.ds(q * L, L)] = (
                        rows_c[j + u, pl.ds(q * L, L)] * pj)
            return 0
        lax.fori_loop(0, CH // 4, _scale, 0)
        pltpu.sync_copy(rows_c, acc_sh.at[dst_c], add=True)
        pltpu.sync_copy(p_v, den_sh.at[dst_c], add=True)
        nb = base + (k + 2) * CH
        pltpu.async_copy(src_hbm.at[pl.ds(nb, CH)], src_c, cur[4])
        pltpu.async_copy(dst_hbm.at[pl.ds(nb, CH)], dst_c, cur[5])

    pltpu.sync_copy(src_hbm.at[pl.ds(base, CH)], src_v0)
    pltpu.sync_copy(dst_hbm.at[pl.ds(base, CH)], dst_v0)
    pltpu.async_copy(src_hbm.at[pl.ds(base + CH, CH)], src_v1, si1)
    pltpu.async_copy(dst_hbm.at[pl.ds(base + CH, CH)], dst_v1, di1)
    pltpu.async_copy(h_hbm.at[src_v0], rows_v0, gsem0)

    def _pair_steps(t, _):
        _process(2 * t, bufs[0], bufs[1])
        _process(2 * t + 1, bufs[1], bufs[0])
        return 0
    lax.fori_loop(0, NCHUNK // 2, _pair_steps, 0)

    pltpu.make_async_copy(src_hbm.at[pl.ds(0, CH)], rows_v0, gsem0).wait()
    pltpu.make_async_copy(src_hbm.at[pl.ds(0, CH)], src_v1, si1).wait()
    pltpu.make_async_copy(dst_hbm.at[pl.ds(0, CH)], dst_v1, di1).wait()
    plsc.subcore_barrier()

    for t in range(ROWS_PER_SUB // CH):
        o = sid * ROWS_PER_SUB + t * CH
        pltpu.sync_copy(acc_sh.at[pl.ds(o, CH)], out_hbm.at[cid, pl.ds(o, CH)])
        pltpu.sync_copy(den_sh.at[pl.ds(o, CH)], den_hbm.at[cid, pl.ds(o, CH)])


@functools.cache
def _sc_conv_kernel():
  return pl.kernel(
    _sc_conv_body,
    out_type=(jax.ShapeDtypeStruct((NC, N_ACC, H), jnp.float32),
              jax.ShapeDtypeStruct((NC, N_ACC), jnp.float32)),
    mesh=_sc_mesh(),
    compiler_params=pltpu.CompilerParams(needs_layout_passes=False, use_tc_tiling_on_sc=False),
    scratch_types=[
        pltpu.VMEM((N,), jnp.float32),
        pltpu.VMEM((N,), jnp.float32),
        pltpu.VMEM((L,), jnp.float32),
        pltpu.VMEM((CH,), jnp.int32),
        pltpu.VMEM((CH,), jnp.int32),
        pltpu.VMEM((CH, H), jnp.float32),
        pltpu.VMEM((CH,), jnp.int32),
        pltpu.VMEM((CH,), jnp.int32),
        pltpu.VMEM((CH, H), jnp.float32),
        pltpu.VMEM((CH,), jnp.float32),
        pltpu.VMEM((CH, H), jnp.float32),
        pltpu.VMEM((CH,), jnp.float32),
        pltpu.VMEM_SHARED((N_ACC, H), jnp.float32),
        pltpu.VMEM_SHARED((N_ACC,), jnp.float32),
        pltpu.SemaphoreType.DMA,
        pltpu.SemaphoreType.DMA,
        pltpu.SemaphoreType.DMA,
        pltpu.SemaphoreType.DMA,
        pltpu.SemaphoreType.DMA,
        pltpu.SemaphoreType.DMA,
    ],
  )


def _sc_pair_body(p_hbm, q_hbm, row_hbm, col_hbm, s_hbm,
                  row_v0, col_v0, pbuf0, qbuf0, sbuf0,
                  row_v1, col_v1, pbuf1, qbuf1, sbuf1,
                  gp0, gq0, gp1, gq1, ri0, ci0, ri1, ci1, wsem0, wsem1):
    cid = lax.axis_index("c")
    sid = lax.axis_index("s")
    wid = sid * NC + cid
    base = wid * PER_W2
    bufs = ((row_v0, col_v0, pbuf0, qbuf0, sbuf0, gp0, gq0, ri0, ci0, wsem0),
            (row_v1, col_v1, pbuf1, qbuf1, sbuf1, gp1, gq1, ri1, ci1, wsem1))

    def _process(k, cur, nxt):
        row_c, col_c, pb_c, qb_c, sb_c, gp_c, gq_c, ri_c, ci_c, ws_c = cur
        row_n, col_n, pb_n, qb_n, sb_n, gp_n, gq_n, ri_n, ci_n, ws_n = nxt
        pltpu.make_async_copy(row_hbm.at[pl.ds(0, CH)], row_n, ri_n).wait()
        pltpu.make_async_copy(col_hbm.at[pl.ds(0, CH)], col_n, ci_n).wait()
        pltpu.async_copy(p_hbm.at[row_n], pb_n, gp_n)
        pltpu.async_copy(q_hbm.at[col_n], qb_n, gq_n)
        pltpu.make_async_copy(row_hbm.at[pl.ds(0, CH)], pb_c, gp_c).wait()
        pltpu.make_async_copy(row_hbm.at[pl.ds(0, CH)], qb_c, gq_c).wait()

        @pl.when(k >= 2)
        def _():
            pltpu.make_async_copy(row_hbm.at[pl.ds(0, CH)], sb_c, ws_c).wait()

        def _add(i, _):
            for q in range(4):
                sb_c[i, pl.ds(q * L, L)] = (pb_c[i, pl.ds(q * L, L)]
                                            + qb_c[i, pl.ds(q * L, L)])
            return 0
        lax.fori_loop(0, CH, _add, 0)
        cb = base + k * CH
        pltpu.async_copy(sb_c, s_hbm.at[pl.ds(cb, CH)], ws_c)
        nb = base + (k + 2) * CH
        pltpu.async_copy(row_hbm.at[pl.ds(nb, CH)], row_c, ri_c)
        pltpu.async_copy(col_hbm.at[pl.ds(nb, CH)], col_c, ci_c)

    pltpu.sync_copy(row_hbm.at[pl.ds(base, CH)], row_v0)
    pltpu.sync_copy(col_hbm.at[pl.ds(base, CH)], col_v0)
    pltpu.async_copy(row_hbm.at[pl.ds(base + CH, CH)], row_v1, ri1)
    pltpu.async_copy(col_hbm.at[pl.ds(base + CH, CH)], col_v1, ci1)
    pltpu.async_copy(p_hbm.at[row_v0], pbuf0, gp0)
    pltpu.async_copy(q_hbm.at[col_v0], qbuf0, gq0)

    def _pair_steps(t, _):
        _process(2 * t, bufs[0], bufs[1])
        _process(2 * t + 1, bufs[1], bufs[0])
        return 0
    lax.fori_loop(0, NCHUNK2 // 2, _pair_steps, 0)

    pltpu.make_async_copy(row_hbm.at[pl.ds(0, CH)], pbuf0, gp0).wait()
    pltpu.make_async_copy(row_hbm.at[pl.ds(0, CH)], qbuf0, gq0).wait()
    pltpu.make_async_copy(row_hbm.at[pl.ds(0, CH)], row_v1, ri1).wait()
    pltpu.make_async_copy(col_hbm.at[pl.ds(0, CH)], col_v1, ci1).wait()
    pltpu.make_async_copy(row_hbm.at[pl.ds(0, CH)], sbuf0, wsem0).wait()
    pltpu.make_async_copy(row_hbm.at[pl.ds(0, CH)], sbuf1, wsem1).wait()


@functools.cache
def _sc_pair_kernel():
  return pl.kernel(
    _sc_pair_body,
    out_type=jax.ShapeDtypeStruct((E2_PAD, 2 * H), jnp.float32),
    mesh=_sc_mesh(),
    compiler_params=pltpu.CompilerParams(needs_layout_passes=False, use_tc_tiling_on_sc=False),
    scratch_types=(
        [pltpu.VMEM((CH,), jnp.int32), pltpu.VMEM((CH,), jnp.int32),
         pltpu.VMEM((CH, H), jnp.float32), pltpu.VMEM((CH, H), jnp.float32),
         pltpu.VMEM((CH, 2 * H), jnp.float32)] * 2
        + [pltpu.SemaphoreType.DMA] * 10
    ),
  )


_BLK = 1000
_NBLK = N // _BLK


def _tc1_body(x_ref, w_ref, as_ref, ad_ref,
              h_ref, asrc_ref, adst_ref, gmax_ref):
    i = pl.program_id(0)
    h = jnp.dot(x_ref[...], w_ref[...], preferred_element_type=jnp.float32)
    h_ref[...] = h
    a_s = jnp.sum(h * as_ref[...], axis=1, keepdims=True)
    a_d = jnp.sum(h * ad_ref[...], axis=1, keepdims=True)
    asrc_ref[...] = a_s
    adst_ref[...] = a_d
    bm = jnp.max(a_s)

    bm2 = bm.reshape(1, 1)

    @pl.when(i == 0)
    def _():
        gmax_ref[...] = bm2

    @pl.when(i > 0)
    def _():
        gmax_ref[...] = jnp.maximum(gmax_ref[...], bm2)


def _tc_mid_body(part_ref, den_ref, b_ref, w_ref, as_ref, ad_ref,
                 h_ref, asrc_ref, adst_ref, gmax_ref):
    i = pl.program_id(0)
    agg = part_ref[0] + part_ref[1]
    den = den_ref[:, 0:1] + den_ref[:, 1:2]
    out = agg / (den + 1e-16) + b_ref[...]
    hx = jnp.maximum(out, 0.0)
    h2 = jnp.dot(hx, w_ref[...], preferred_element_type=jnp.float32)
    h_ref[...] = h2
    a_s = jnp.sum(h2 * as_ref[...], axis=1, keepdims=True)
    a_d = jnp.sum(h2 * ad_ref[...], axis=1, keepdims=True)
    asrc_ref[...] = a_s
    adst_ref[...] = a_d
    bm = jnp.max(a_s)

    bm2 = bm.reshape(1, 1)

    @pl.when(i == 0)
    def _():
        gmax_ref[...] = bm2

    @pl.when(i > 0)
    def _():
        gmax_ref[...] = jnp.maximum(gmax_ref[...], bm2)


def _tc3_body(part_ref, den_ref, b_ref, wa_ref, wb_ref, mb_ref,
              p_ref, q_ref):
    agg = part_ref[0] + part_ref[1]
    den = den_ref[:, 0:1] + den_ref[:, 1:2]
    hf = agg / (den + 1e-16) + b_ref[...]
    p_ref[...] = (jnp.dot(hf, wa_ref[...], preferred_element_type=jnp.float32)
                  + mb_ref[...])
    q_ref[...] = jnp.dot(hf, wb_ref[...], preferred_element_type=jnp.float32)


_EBLK = 2000
_NEBLK = EH // _EBLK
_LOG_PRIOR = float(np.log(np.float32(1.0 / 3.0) + np.float32(1e-12)))


def _tc4_body(s_ref, w_ref, b_ref,
              logits_ref, probs_ref, kl_ref, rec_ref):
    i = pl.program_id(0)
    hid = jnp.maximum(s_ref[:, :H], 0.0)
    lf = jnp.dot(hid, w_ref[...], preferred_element_type=jnp.float32) + b_ref[...]
    col = lax.broadcasted_iota(jnp.int32, lf.shape, 1)
    valid = col < 3
    lfm = jnp.where(valid, lf, -jnp.inf)
    m = jnp.max(lfm, axis=1, keepdims=True)
    e = jnp.where(valid, jnp.exp(lf - m), 0.0)
    den = jnp.sum(e, axis=1, keepdims=True)
    p = e / den
    logits_ref[...] = lf[:, 0:3]
    probs_ref[...] = p[:, 0:3]
    lp = jnp.log(jnp.where(valid, p, 1.0))
    kl = jnp.where(valid, p * (lp - _LOG_PRIOR), 0.0)
    p02 = jnp.sum(jnp.where(col == 1, 0.0, e), axis=1, keepdims=True) / den
    rec = jnp.log(p02 + 1e-12)
    kl_s = jnp.sum(kl)
    rec_s = jnp.sum(rec)

    kl_s2 = kl_s.reshape(1, 1)
    rec_s2 = rec_s.reshape(1, 1)

    @pl.when(i == 0)
    def _():
        kl_ref[...] = kl_s2
        rec_ref[...] = rec_s2

    @pl.when(i > 0)
    def _():
        kl_ref[...] = kl_ref[...] + kl_s2
        rec_ref[...] = rec_ref[...] + rec_s2


def _node_stage1(x, W1, att_src1, att_dst1):
    return pl.pallas_call(
        _tc1_body,
        grid=(_NBLK,),
        in_specs=[
            pl.BlockSpec((_BLK, D), lambda i: (i, 0)),
            pl.BlockSpec((D, H), lambda i: (0, 0)),
            pl.BlockSpec((1, H), lambda i: (0, 0)),
            pl.BlockSpec((1, H), lambda i: (0, 0)),
        ],
        out_specs=[
            pl.BlockSpec((_BLK, H), lambda i: (i, 0)),
            pl.BlockSpec((_BLK, 1), lambda i: (i, 0)),
            pl.BlockSpec((_BLK, 1), lambda i: (i, 0)),
            pl.BlockSpec((1, 1), lambda i: (0, 0)),
        ],
        out_shape=[
            jax.ShapeDtypeStruct((N, H), jnp.float32),
            jax.ShapeDtypeStruct((N, 1), jnp.float32),
            jax.ShapeDtypeStruct((N, 1), jnp.float32),
            jax.ShapeDtypeStruct((1, 1), jnp.float32),
        ],
    )(x, W1, att_src1.reshape(1, H), att_dst1.reshape(1, H))


def _node_stage2(part, den_t, bias1, W2, att_src2, att_dst2):
    return pl.pallas_call(
        _tc_mid_body,
        grid=(_NBLK,),
        in_specs=[
            pl.BlockSpec((NC, _BLK, H), lambda i: (0, i, 0)),
            pl.BlockSpec((_BLK, NC), lambda i: (i, 0)),
            pl.BlockSpec((1, H), lambda i: (0, 0)),
            pl.BlockSpec((H, H), lambda i: (0, 0)),
            pl.BlockSpec((1, H), lambda i: (0, 0)),
            pl.BlockSpec((1, H), lambda i: (0, 0)),
        ],
        out_specs=[
            pl.BlockSpec((_BLK, H), lambda i: (i, 0)),
            pl.BlockSpec((_BLK, 1), lambda i: (i, 0)),
            pl.BlockSpec((_BLK, 1), lambda i: (i, 0)),
            pl.BlockSpec((1, 1), lambda i: (0, 0)),
        ],
        out_shape=[
            jax.ShapeDtypeStruct((N, H), jnp.float32),
            jax.ShapeDtypeStruct((N, 1), jnp.float32),
            jax.ShapeDtypeStruct((N, 1), jnp.float32),
            jax.ShapeDtypeStruct((1, 1), jnp.float32),
        ],
    )(part, den_t, bias1.reshape(1, H), W2,
      att_src2.reshape(1, H), att_dst2.reshape(1, H))


def _node_stage3(part, den_t, bias2, mW1, mb1):
    return pl.pallas_call(
        _tc3_body,
        grid=(_NBLK,),
        in_specs=[
            pl.BlockSpec((NC, _BLK, H), lambda i: (0, i, 0)),
            pl.BlockSpec((_BLK, NC), lambda i: (i, 0)),
            pl.BlockSpec((1, H), lambda i: (0, 0)),
            pl.BlockSpec((H, H), lambda i: (0, 0)),
            pl.BlockSpec((H, H), lambda i: (0, 0)),
            pl.BlockSpec((1, H), lambda i: (0, 0)),
        ],
        out_specs=[
            pl.BlockSpec((_BLK, H), lambda i: (i, 0)),
            pl.BlockSpec((_BLK, H), lambda i: (i, 0)),
        ],
        out_shape=[
            jax.ShapeDtypeStruct((N, H), jnp.float32),
            jax.ShapeDtypeStruct((N, H), jnp.float32),
        ],
    )(part, den_t, bias2.reshape(1, H), mW1[:H], mW1[H:], mb1.reshape(1, H))


def _edge_stage(S2, mW2, mb2):
    w_pad = jnp.zeros((H, 128), jnp.float32).at[:, :3].set(mW2)
    b_pad = jnp.zeros((1, 128), jnp.float32).at[0, :3].set(mb2)
    return pl.pallas_call(
        _tc4_body,
        grid=(_NEBLK,),
        in_specs=[
            pl.BlockSpec((_EBLK, 2 * H), lambda i: (i, 0)),
            pl.BlockSpec((H, 128), lambda i: (0, 0)),
            pl.BlockSpec((1, 128), lambda i: (0, 0)),
        ],
        out_specs=[
            pl.BlockSpec((_EBLK, 3), lambda i: (i, 0)),
            pl.BlockSpec((_EBLK, 3), lambda i: (i, 0)),
            pl.BlockSpec((1, 1), lambda i: (0, 0)),
            pl.BlockSpec((1, 1), lambda i: (0, 0)),
        ],
        out_shape=[
            jax.ShapeDtypeStruct((EH, 3), jnp.float32),
            jax.ShapeDtypeStruct((EH, 3), jnp.float32),
            jax.ShapeDtypeStruct((1, 1), jnp.float32),
            jax.ShapeDtypeStruct((1, 1), jnp.float32),
        ],
    )(S2, w_pad, b_pad)


def kernel(x, edge_index, W1, att_src1, att_dst1, bias1,
           W2, att_src2, att_dst2, bias2, mW1, mb1, mW2, mb2):
    src = edge_index[0]
    dst = edge_index[1]
    loop_idx = jnp.arange(N, dtype=jnp.int32)

    n_dummy = E_IDX - EP
    src_full = jnp.concatenate(
        [src, loop_idx, jnp.arange(n_dummy, dtype=jnp.int32) % N])
    dst_full = jnp.concatenate(
        [dst, loop_idx, jnp.zeros((n_dummy,), jnp.int32)])

    n_d2 = E2_IDX - EH
    d2 = jnp.arange(n_d2, dtype=jnp.int32) % N
    row_a = jnp.concatenate([src[:EH], d2])
    col_a = jnp.concatenate([dst[:EH], d2])
    row_b = jnp.concatenate([src[EH:], d2])
    col_b = jnp.concatenate([dst[EH:], d2])

    h1, as1, ad1, gm1 = _node_stage1(x, W1, att_src1, att_dst1)
    gvec1 = jnp.broadcast_to(gm1.reshape(()), (L,))
    part1, den1 = _sc_conv_kernel()(h1, as1.reshape(N), ad1.reshape(N), gvec1,
                                    src_full, dst_full)

    h2, as2, ad2, gm2 = _node_stage2(part1, den1.T, bias1, W2,
                                     att_src2, att_dst2)
    gvec2 = jnp.broadcast_to(gm2.reshape(()), (L,))
    part2, den2 = _sc_conv_kernel()(h2, as2.reshape(N), ad2.reshape(N), gvec2,
                                    src_full, dst_full)

    P, Q = _node_stage3(part2, den2.T, bias2, mW1, mb1)
    s_a = _sc_pair_kernel()(P, Q, row_a, col_a)
    s_b = _sc_pair_kernel()(P, Q, row_b, col_b)
    lg_a, pr_a, kl_a, rec_a = _edge_stage(s_a, mW2, mb2)
    lg_b, pr_b, kl_b, rec_b = _edge_stage(s_b, mW2, mb2)
    logits = jnp.concatenate([lg_a, lg_b], axis=0)
    probs = jnp.concatenate([pr_a, pr_b], axis=0)
    kl_sum = kl_a + kl_b
    rec_sum = rec_a + rec_b

    struct_loss = (kl_sum.reshape(()) - rec_sum.reshape(())) / jnp.float32(E)
    return (logits, probs, struct_loss)


# speedup vs baseline: 1.2383x; 1.0083x over previous
import functools

import jax
import jax.numpy as jnp
import numpy as np
from jax import lax
from jax.experimental import pallas as pl
from jax.experimental.pallas import tpu as pltpu
from jax.experimental.pallas import tpu_sc as plsc

N, E, D, H = 10000, 320000, 128, 64
NC, NS, L = 2, 16, 16
NW = NC * NS
CH = 128

EP = E + N
NCHUNK = 82
PER_W = NCHUNK * CH
E_PAD = NW * PER_W
E_IDX = E_PAD + 2 * CH

NSPLIT = 4
EH = E // NSPLIT
NCHUNK2 = 20
PER_W2 = NCHUNK2 * CH
E2_PAD = NW * PER_W2
E2_IDX = E2_PAD + 2 * CH

N_ACC = 10240
ROWS_PER_SUB = N_ACC // NS

@functools.cache
def _sc_mesh():
    return plsc.VectorSubcoreMesh(core_axis_name="c", subcore_axis_name="s",
                                  num_cores=NC, num_subcores=NS)


def _sc_conv_body(h_hbm, asrc_hbm, adst_hbm, gmax_hbm, src_hbm, dst_hbm,
                  out_hbm, den_hbm,
                  asrc_v, adst_v, gmax_v,
                  src_v0, dst_v0, rows_v0, src_v1, dst_v1, rows_v1, p_v,
                  zbuf, zden, acc_sh, den_sh,
                  gsem0, gsem1, si0, di0, si1, di1):
    cid = lax.axis_index("c")
    sid = lax.axis_index("s")
    wid = sid * NC + cid

    pltpu.sync_copy(asrc_hbm, asrc_v)
    pltpu.sync_copy(adst_hbm, adst_v)
    pltpu.sync_copy(gmax_hbm, gmax_v)

    def _zrow(i, _):
        for q in range(4):
            zbuf[i, pl.ds(q * L, L)] = jnp.zeros((L,), jnp.float32)
        return 0
    lax.fori_loop(0, CH, _zrow, 0)

    def _zden(i, _):
        zden[pl.ds(i * L, L)] = jnp.zeros((L,), jnp.float32)
        return 0
    lax.fori_loop(0, CH // L, _zden, 0)

    for t in range(ROWS_PER_SUB // CH):
        pltpu.sync_copy(zbuf, acc_sh.at[pl.ds(sid * ROWS_PER_SUB + t * CH, CH)])
        pltpu.sync_copy(zden, den_sh.at[pl.ds(sid * ROWS_PER_SUB + t * CH, CH)])
    plsc.subcore_barrier()

    base = wid * PER_W
    bufs = ((src_v0, dst_v0, rows_v0, gsem0, si0, di0),
            (src_v1, dst_v1, rows_v1, gsem1, si1, di1))

    def _process(k, cur, nxt):
        src_c, dst_c, rows_c, gsem_c, _, _ = cur
        src_n, dst_n, rows_n, gsem_n, si_n, di_n = nxt
        pltpu.make_async_copy(src_hbm.at[pl.ds(0, CH)], src_n, si_n).wait()
        pltpu.make_async_copy(dst_hbm.at[pl.ds(0, CH)], dst_n, di_n).wait()
        pltpu.async_copy(h_hbm.at[src_n], rows_n, gsem_n)
        cb = base + k * CH
        gvec = gmax_v[...]
        for g in range(CH // L):
            s_idx = src_c[pl.ds(g * L, L)]
            d_idx = dst_c[pl.ds(g * L, L)]
            a_s = plsc.load_gather(asrc_v, [s_idx])
            a_d = plsc.load_gather(adst_v, [d_idx])
            al = a_s + a_d
            al = jnp.where(al >= 0.0, al, 0.2 * al)
            m = gvec + a_d
            m = jnp.where(m >= 0.0, m, 0.2 * m)
            p = jnp.exp(al - m)
            pos = cb + g * L + lax.iota(jnp.int32, L)
            p = jnp.where(pos < EP, p, 0.0)
            p_v[pl.ds(g * L, L)] = p
        pltpu.make_async_copy(src_hbm.at[pl.ds(0, CH)], rows_c, gsem_c).wait()

        def _scale(j4, _):
            j = j4 * 4
            for u in range(4):
                pj = plsc.load_gather(p_v, [jnp.full((L,), j + u, jnp.int32)])
                for q in range(4):
                    rows_c[j + u, pl.ds(q * L, L)] = (
                        rows_c[j + u, pl.ds(q * L, L)] * pj)
            return 0
        lax.fori_loop(0, CH // 4, _scale, 0)
        pltpu.sync_copy(rows_c, acc_sh.at[dst_c], add=True)
        pltpu.sync_copy(p_v, den_sh.at[dst_c], add=True)
        nb = base + (k + 2) * CH
        pltpu.async_copy(src_hbm.at[pl.ds(nb, CH)], src_c, cur[4])
        pltpu.async_copy(dst_hbm.at[pl.ds(nb, CH)], dst_c, cur[5])

    pltpu.sync_copy(src_hbm.at[pl.ds(base, CH)], src_v0)
    pltpu.sync_copy(dst_hbm.at[pl.ds(base, CH)], dst_v0)
    pltpu.async_copy(src_hbm.at[pl.ds(base + CH, CH)], src_v1, si1)
    pltpu.async_copy(dst_hbm.at[pl.ds(base + CH, CH)], dst_v1, di1)
    pltpu.async_copy(h_hbm.at[src_v0], rows_v0, gsem0)

    def _pair_steps(t, _):
        _process(2 * t, bufs[0], bufs[1])
        _process(2 * t + 1, bufs[1], bufs[0])
        return 0
    lax.fori_loop(0, NCHUNK // 2, _pair_steps, 0)

    pltpu.make_async_copy(src_hbm.at[pl.ds(0, CH)], rows_v0, gsem0).wait()
    pltpu.make_async_copy(src_hbm.at[pl.ds(0, CH)], src_v1, si1).wait()
    pltpu.make_async_copy(dst_hbm.at[pl.ds(0, CH)], dst_v1, di1).wait()
    plsc.subcore_barrier()

    for t in range(ROWS_PER_SUB // CH):
        o = sid * ROWS_PER_SUB + t * CH
        pltpu.sync_copy(acc_sh.at[pl.ds(o, CH)], out_hbm.at[cid, pl.ds(o, CH)])
        pltpu.sync_copy(den_sh.at[pl.ds(o, CH)], den_hbm.at[cid, pl.ds(o, CH)])


@functools.cache
def _sc_conv_kernel():
  return pl.kernel(
    _sc_conv_body,
    out_type=(jax.ShapeDtypeStruct((NC, N_ACC, H), jnp.float32),
              jax.ShapeDtypeStruct((NC, N_ACC), jnp.float32)),
    mesh=_sc_mesh(),
    compiler_params=pltpu.CompilerParams(needs_layout_passes=False, use_tc_tiling_on_sc=False),
    scratch_types=[
        pltpu.VMEM((N,), jnp.float32),
        pltpu.VMEM((N,), jnp.float32),
        pltpu.VMEM((L,), jnp.float32),
        pltpu.VMEM((CH,), jnp.int32),
        pltpu.VMEM((CH,), jnp.int32),
        pltpu.VMEM((CH, H), jnp.float32),
        pltpu.VMEM((CH,), jnp.int32),
        pltpu.VMEM((CH,), jnp.int32),
        pltpu.VMEM((CH, H), jnp.float32),
        pltpu.VMEM((CH,), jnp.float32),
        pltpu.VMEM((CH, H), jnp.float32),
        pltpu.VMEM((CH,), jnp.float32),
        pltpu.VMEM_SHARED((N_ACC, H), jnp.float32),
        pltpu.VMEM_SHARED((N_ACC,), jnp.float32),
        pltpu.SemaphoreType.DMA,
        pltpu.SemaphoreType.DMA,
        pltpu.SemaphoreType.DMA,
        pltpu.SemaphoreType.DMA,
        pltpu.SemaphoreType.DMA,
        pltpu.SemaphoreType.DMA,
    ],
  )


def _sc_pair_body(p_hbm, q_hbm, row_hbm, col_hbm, s_hbm,
                  row_v0, col_v0, pbuf0, qbuf0, sbuf0,
                  row_v1, col_v1, pbuf1, qbuf1, sbuf1,
                  gp0, gq0, gp1, gq1, ri0, ci0, ri1, ci1, wsem0, wsem1):
    cid = lax.axis_index("c")
    sid = lax.axis_index("s")
    wid = sid * NC + cid
    base = wid * PER_W2
    bufs = ((row_v0, col_v0, pbuf0, qbuf0, sbuf0, gp0, gq0, ri0, ci0, wsem0),
            (row_v1, col_v1, pbuf1, qbuf1, sbuf1, gp1, gq1, ri1, ci1, wsem1))

    def _process(k, cur, nxt):
        row_c, col_c, pb_c, qb_c, sb_c, gp_c, gq_c, ri_c, ci_c, ws_c = cur
        row_n, col_n, pb_n, qb_n, sb_n, gp_n, gq_n, ri_n, ci_n, ws_n = nxt
        pltpu.make_async_copy(row_hbm.at[pl.ds(0, CH)], row_n, ri_n).wait()
        pltpu.make_async_copy(col_hbm.at[pl.ds(0, CH)], col_n, ci_n).wait()
        pltpu.async_copy(p_hbm.at[row_n], pb_n, gp_n)
        pltpu.async_copy(q_hbm.at[col_n], qb_n, gq_n)
        pltpu.make_async_copy(row_hbm.at[pl.ds(0, CH)], pb_c, gp_c).wait()
        pltpu.make_async_copy(row_hbm.at[pl.ds(0, CH)], qb_c, gq_c).wait()

        @pl.when(k >= 2)
        def _():
            pltpu.make_async_copy(row_hbm.at[pl.ds(0, CH)], sb_c, ws_c).wait()

        def _add(i, _):
            for q in range(4):
                sb_c[i, pl.ds(q * L, L)] = (pb_c[i, pl.ds(q * L, L)]
                                            + qb_c[i, pl.ds(q * L, L)])
            return 0
        lax.fori_loop(0, CH, _add, 0)
        cb = base + k * CH
        pltpu.async_copy(sb_c, s_hbm.at[pl.ds(cb, CH)], ws_c)
        nb = base + (k + 2) * CH
        pltpu.async_copy(row_hbm.at[pl.ds(nb, CH)], row_c, ri_c)
        pltpu.async_copy(col_hbm.at[pl.ds(nb, CH)], col_c, ci_c)

    pltpu.sync_copy(row_hbm.at[pl.ds(base, CH)], row_v0)
    pltpu.sync_copy(col_hbm.at[pl.ds(base, CH)], col_v0)
    pltpu.async_copy(row_hbm.at[pl.ds(base + CH, CH)], row_v1, ri1)
    pltpu.async_copy(col_hbm.at[pl.ds(base + CH, CH)], col_v1, ci1)
    pltpu.async_copy(p_hbm.at[row_v0], pbuf0, gp0)
    pltpu.async_copy(q_hbm.at[col_v0], qbuf0, gq0)

    def _pair_steps(t, _):
        _process(2 * t, bufs[0], bufs[1])
        _process(2 * t + 1, bufs[1], bufs[0])
        return 0
    lax.fori_loop(0, NCHUNK2 // 2, _pair_steps, 0)

    pltpu.make_async_copy(row_hbm.at[pl.ds(0, CH)], pbuf0, gp0).wait()
    pltpu.make_async_copy(row_hbm.at[pl.ds(0, CH)], qbuf0, gq0).wait()
    pltpu.make_async_copy(row_hbm.at[pl.ds(0, CH)], row_v1, ri1).wait()
    pltpu.make_async_copy(col_hbm.at[pl.ds(0, CH)], col_v1, ci1).wait()
    pltpu.make_async_copy(row_hbm.at[pl.ds(0, CH)], sbuf0, wsem0).wait()
    pltpu.make_async_copy(row_hbm.at[pl.ds(0, CH)], sbuf1, wsem1).wait()


@functools.cache
def _sc_pair_kernel():
  return pl.kernel(
    _sc_pair_body,
    out_type=jax.ShapeDtypeStruct((E2_PAD, 2 * H), jnp.float32),
    mesh=_sc_mesh(),
    compiler_params=pltpu.CompilerParams(needs_layout_passes=False, use_tc_tiling_on_sc=False),
    scratch_types=(
        [pltpu.VMEM((CH,), jnp.int32), pltpu.VMEM((CH,), jnp.int32),
         pltpu.VMEM((CH, H), jnp.float32), pltpu.VMEM((CH, H), jnp.float32),
         pltpu.VMEM((CH, 2 * H), jnp.float32)] * 2
        + [pltpu.SemaphoreType.DMA] * 10
    ),
  )


_BLK = 1000
_NBLK = N // _BLK


def _tc1_body(x_ref, w_ref, as_ref, ad_ref,
              h_ref, asrc_ref, adst_ref, gmax_ref):
    i = pl.program_id(0)
    h = jnp.dot(x_ref[...], w_ref[...], preferred_element_type=jnp.float32)
    h_ref[...] = h
    a_s = jnp.sum(h * as_ref[...], axis=1, keepdims=True)
    a_d = jnp.sum(h * ad_ref[...], axis=1, keepdims=True)
    asrc_ref[...] = a_s
    adst_ref[...] = a_d
    bm = jnp.max(a_s)

    bm2 = bm.reshape(1, 1)

    @pl.when(i == 0)
    def _():
        gmax_ref[...] = bm2

    @pl.when(i > 0)
    def _():
        gmax_ref[...] = jnp.maximum(gmax_ref[...], bm2)


def _tc_mid_body(part_ref, den_ref, b_ref, w_ref, as_ref, ad_ref,
                 h_ref, asrc_ref, adst_ref, gmax_ref):
    i = pl.program_id(0)
    agg = part_ref[0] + part_ref[1]
    den = den_ref[:, 0:1] + den_ref[:, 1:2]
    out = agg / (den + 1e-16) + b_ref[...]
    hx = jnp.maximum(out, 0.0)
    h2 = jnp.dot(hx, w_ref[...], preferred_element_type=jnp.float32)
    h_ref[...] = h2
    a_s = jnp.sum(h2 * as_ref[...], axis=1, keepdims=True)
    a_d = jnp.sum(h2 * ad_ref[...], axis=1, keepdims=True)
    asrc_ref[...] = a_s
    adst_ref[...] = a_d
    bm = jnp.max(a_s)

    bm2 = bm.reshape(1, 1)

    @pl.when(i == 0)
    def _():
        gmax_ref[...] = bm2

    @pl.when(i > 0)
    def _():
        gmax_ref[...] = jnp.maximum(gmax_ref[...], bm2)


def _tc3_body(part_ref, den_ref, b_ref, wa_ref, wb_ref, mb_ref,
              p_ref, q_ref):
    agg = part_ref[0] + part_ref[1]
    den = den_ref[:, 0:1] + den_ref[:, 1:2]
    hf = agg / (den + 1e-16) + b_ref[...]
    p_ref[...] = (jnp.dot(hf, wa_ref[...], preferred_element_type=jnp.float32)
                  + mb_ref[...])
    q_ref[...] = jnp.dot(hf, wb_ref[...], preferred_element_type=jnp.float32)


_EBLK = 2000
_NEBLK = EH // _EBLK
_LOG_PRIOR = float(np.log(np.float32(1.0 / 3.0) + np.float32(1e-12)))


def _tc4_body(s_ref, w_ref, b_ref,
              logits_ref, probs_ref, kl_ref, rec_ref):
    i = pl.program_id(0)
    hid = jnp.maximum(s_ref[:, :H], 0.0)
    lf = jnp.dot(hid, w_ref[...], preferred_element_type=jnp.float32) + b_ref[...]
    col = lax.broadcasted_iota(jnp.int32, lf.shape, 1)
    valid = col < 3
    lfm = jnp.where(valid, lf, -jnp.inf)
    m = jnp.max(lfm, axis=1, keepdims=True)
    e = jnp.where(valid, jnp.exp(lf - m), 0.0)
    den = jnp.sum(e, axis=1, keepdims=True)
    p = e / den
    logits_ref[...] = lf[:, 0:3]
    probs_ref[...] = p[:, 0:3]
    lp = jnp.log(jnp.where(valid, p, 1.0))
    kl = jnp.where(valid, p * (lp - _LOG_PRIOR), 0.0)
    p02 = jnp.sum(jnp.where(col == 1, 0.0, e), axis=1, keepdims=True) / den
    rec = jnp.log(p02 + 1e-12)
    kl_s = jnp.sum(kl)
    rec_s = jnp.sum(rec)

    kl_s2 = kl_s.reshape(1, 1)
    rec_s2 = rec_s.reshape(1, 1)

    @pl.when(i == 0)
    def _():
        kl_ref[...] = kl_s2
        rec_ref[...] = rec_s2

    @pl.when(i > 0)
    def _():
        kl_ref[...] = kl_ref[...] + kl_s2
        rec_ref[...] = rec_ref[...] + rec_s2


def _node_stage1(x, W1, att_src1, att_dst1):
    return pl.pallas_call(
        _tc1_body,
        grid=(_NBLK,),
        in_specs=[
            pl.BlockSpec((_BLK, D), lambda i: (i, 0)),
            pl.BlockSpec((D, H), lambda i: (0, 0)),
            pl.BlockSpec((1, H), lambda i: (0, 0)),
            pl.BlockSpec((1, H), lambda i: (0, 0)),
        ],
        out_specs=[
            pl.BlockSpec((_BLK, H), lambda i: (i, 0)),
            pl.BlockSpec((_BLK, 1), lambda i: (i, 0)),
            pl.BlockSpec((_BLK, 1), lambda i: (i, 0)),
            pl.BlockSpec((1, 1), lambda i: (0, 0)),
        ],
        out_shape=[
            jax.ShapeDtypeStruct((N, H), jnp.float32),
            jax.ShapeDtypeStruct((N, 1), jnp.float32),
            jax.ShapeDtypeStruct((N, 1), jnp.float32),
            jax.ShapeDtypeStruct((1, 1), jnp.float32),
        ],
    )(x, W1, att_src1.reshape(1, H), att_dst1.reshape(1, H))


def _node_stage2(part, den_t, bias1, W2, att_src2, att_dst2):
    return pl.pallas_call(
        _tc_mid_body,
        grid=(_NBLK,),
        in_specs=[
            pl.BlockSpec((NC, _BLK, H), lambda i: (0, i, 0)),
            pl.BlockSpec((_BLK, NC), lambda i: (i, 0)),
            pl.BlockSpec((1, H), lambda i: (0, 0)),
            pl.BlockSpec((H, H), lambda i: (0, 0)),
            pl.BlockSpec((1, H), lambda i: (0, 0)),
            pl.BlockSpec((1, H), lambda i: (0, 0)),
        ],
        out_specs=[
            pl.BlockSpec((_BLK, H), lambda i: (i, 0)),
            pl.BlockSpec((_BLK, 1), lambda i: (i, 0)),
            pl.BlockSpec((_BLK, 1), lambda i: (i, 0)),
            pl.BlockSpec((1, 1), lambda i: (0, 0)),
        ],
        out_shape=[
            jax.ShapeDtypeStruct((N, H), jnp.float32),
            jax.ShapeDtypeStruct((N, 1), jnp.float32),
            jax.ShapeDtypeStruct((N, 1), jnp.float32),
            jax.ShapeDtypeStruct((1, 1), jnp.float32),
        ],
    )(part, den_t, bias1.reshape(1, H), W2,
      att_src2.reshape(1, H), att_dst2.reshape(1, H))


def _node_stage3(part, den_t, bias2, mW1, mb1):
    return pl.pallas_call(
        _tc3_body,
        grid=(_NBLK,),
        in_specs=[
            pl.BlockSpec((NC, _BLK, H), lambda i: (0, i, 0)),
            pl.BlockSpec((_BLK, NC), lambda i: (i, 0)),
            pl.BlockSpec((1, H), lambda i: (0, 0)),
            pl.BlockSpec((H, H), lambda i: (0, 0)),
            pl.BlockSpec((H, H), lambda i: (0, 0)),
            pl.BlockSpec((1, H), lambda i: (0, 0)),
        ],
        out_specs=[
            pl.BlockSpec((_BLK, H), lambda i: (i, 0)),
            pl.BlockSpec((_BLK, H), lambda i: (i, 0)),
        ],
        out_shape=[
            jax.ShapeDtypeStruct((N, H), jnp.float32),
            jax.ShapeDtypeStruct((N, H), jnp.float32),
        ],
    )(part, den_t, bias2.reshape(1, H), mW1[:H], mW1[H:], mb1.reshape(1, H))


def _edge_stage(S2, mW2, mb2):
    w_pad = jnp.zeros((H, 128), jnp.float32).at[:, :3].set(mW2)
    b_pad = jnp.zeros((1, 128), jnp.float32).at[0, :3].set(mb2)
    return pl.pallas_call(
        _tc4_body,
        grid=(_NEBLK,),
        in_specs=[
            pl.BlockSpec((_EBLK, 2 * H), lambda i: (i, 0)),
            pl.BlockSpec((H, 128), lambda i: (0, 0)),
            pl.BlockSpec((1, 128), lambda i: (0, 0)),
        ],
        out_specs=[
            pl.BlockSpec((_EBLK, 3), lambda i: (i, 0)),
            pl.BlockSpec((_EBLK, 3), lambda i: (i, 0)),
            pl.BlockSpec((1, 1), lambda i: (0, 0)),
            pl.BlockSpec((1, 1), lambda i: (0, 0)),
        ],
        out_shape=[
            jax.ShapeDtypeStruct((EH, 3), jnp.float32),
            jax.ShapeDtypeStruct((EH, 3), jnp.float32),
            jax.ShapeDtypeStruct((1, 1), jnp.float32),
            jax.ShapeDtypeStruct((1, 1), jnp.float32),
        ],
    )(S2, w_pad, b_pad)


def kernel(x, edge_index, W1, att_src1, att_dst1, bias1,
           W2, att_src2, att_dst2, bias2, mW1, mb1, mW2, mb2):
    src = edge_index[0]
    dst = edge_index[1]
    loop_idx = jnp.arange(N, dtype=jnp.int32)

    n_dummy = E_IDX - EP
    src_full = jnp.concatenate(
        [src, loop_idx, jnp.arange(n_dummy, dtype=jnp.int32) % N])
    dst_full = jnp.concatenate(
        [dst, loop_idx, jnp.zeros((n_dummy,), jnp.int32)])

    n_d2 = E2_IDX - EH
    d2 = jnp.arange(n_d2, dtype=jnp.int32) % N
    rows_q = [jnp.concatenate([src[q * EH:(q + 1) * EH], d2])
              for q in range(NSPLIT)]
    cols_q = [jnp.concatenate([dst[q * EH:(q + 1) * EH], d2])
              for q in range(NSPLIT)]

    h1, as1, ad1, gm1 = _node_stage1(x, W1, att_src1, att_dst1)
    gvec1 = jnp.broadcast_to(gm1.reshape(()), (L,))
    part1, den1 = _sc_conv_kernel()(h1, as1.reshape(N), ad1.reshape(N), gvec1,
                                    src_full, dst_full)

    h2, as2, ad2, gm2 = _node_stage2(part1, den1.T, bias1, W2,
                                     att_src2, att_dst2)
    gvec2 = jnp.broadcast_to(gm2.reshape(()), (L,))
    part2, den2 = _sc_conv_kernel()(h2, as2.reshape(N), ad2.reshape(N), gvec2,
                                    src_full, dst_full)

    P, Q = _node_stage3(part2, den2.T, bias2, mW1, mb1)
    parts = []
    for q in range(NSPLIT):
        s_q = _sc_pair_kernel()(P, Q, rows_q[q], cols_q[q])
        parts.append(_edge_stage(s_q, mW2, mb2))
    logits = jnp.concatenate([t[0] for t in parts], axis=0)
    probs = jnp.concatenate([t[1] for t in parts], axis=0)
    kl_sum = sum(t[2] for t in parts)
    rec_sum = sum(t[3] for t in parts)

    struct_loss = (kl_sum.reshape(()) - rec_sum.reshape(())) / jnp.float32(E)
    return (logits, probs, struct_loss)


# speedup vs baseline: 1.3037x; 1.0529x over previous
import functools

import jax
import jax.numpy as jnp
import numpy as np
from jax import lax
from jax.experimental import pallas as pl
from jax.experimental.pallas import tpu as pltpu
from jax.experimental.pallas import tpu_sc as plsc

N, E, D, H = 10000, 320000, 128, 64
NC, NS, L = 2, 16, 16
NW = NC * NS
CH = 128

EP = E + N
NCHUNK = 82
PER_W = NCHUNK * CH
E_PAD = NW * PER_W
E_IDX = E_PAD + 2 * CH

NSPLIT = 4
EH = E // NSPLIT
NCHUNK2 = 20
PER_W2 = NCHUNK2 * CH
E2_PAD = NW * PER_W2
E2_IDX = E2_PAD + 2 * CH

N_ACC = 10240
ROWS_PER_SUB = N_ACC // NS

@functools.cache
def _sc_mesh():
    return plsc.VectorSubcoreMesh(core_axis_name="c", subcore_axis_name="s",
                                  num_cores=NC, num_subcores=NS)


def _sc_conv_body(h_hbm, asrc_hbm, adst_hbm, gmax_hbm, src_hbm, dst_hbm,
                  out_hbm, den_hbm,
                  asrc_v, adst_v, gmax_v,
                  src_v0, dst_v0, rows_v0, dsc_v0,
                  src_v1, dst_v1, rows_v1, dsc_v1, p_v,
                  zbuf, zden, acc_sh, den_sh,
                  gsem0, gsem1, si0, di0, si1, di1, ssem0, ssem1):
    cid = lax.axis_index("c")
    sid = lax.axis_index("s")
    wid = sid * NC + cid

    pltpu.sync_copy(asrc_hbm, asrc_v)
    pltpu.sync_copy(adst_hbm, adst_v)
    pltpu.sync_copy(gmax_hbm, gmax_v)

    def _zrow(i, _):
        for q in range(4):
            zbuf[i, pl.ds(q * L, L)] = jnp.zeros((L,), jnp.float32)
        return 0
    lax.fori_loop(0, CH, _zrow, 0)

    def _zden(i, _):
        zden[pl.ds(i * L, L)] = jnp.zeros((L,), jnp.float32)
        return 0
    lax.fori_loop(0, CH // L, _zden, 0)

    for t in range(ROWS_PER_SUB // CH):
        pltpu.sync_copy(zbuf, acc_sh.at[pl.ds(sid * ROWS_PER_SUB + t * CH, CH)])
        pltpu.sync_copy(zden, den_sh.at[pl.ds(sid * ROWS_PER_SUB + t * CH, CH)])
    plsc.subcore_barrier()

    base = wid * PER_W
    bufs = ((src_v0, dst_v0, rows_v0, dsc_v0, gsem0, si0, di0, ssem0),
            (src_v1, dst_v1, rows_v1, dsc_v1, gsem1, si1, di1, ssem1))

    def _process(k, cur, nxt):
        src_c, dst_c, rows_c, dsc_c, gsem_c, si_c, di_c, ssem_c = cur
        src_n, dst_n, rows_n, dsc_n, gsem_n, si_n, di_n, ssem_n = nxt
        pltpu.make_async_copy(src_hbm.at[pl.ds(0, CH)], src_n, si_n).wait()
        pltpu.make_async_copy(dst_hbm.at[pl.ds(0, CH)], dst_n, di_n).wait()

        @pl.when(k > 0)
        def _():
            pltpu.make_async_copy(src_hbm.at[pl.ds(0, CH)], rows_n, ssem_n).wait()
        pltpu.async_copy(h_hbm.at[src_n], rows_n, gsem_n)
        cb = base + k * CH
        gvec = gmax_v[...]
        for g in range(CH // L):
            s_idx = src_c[pl.ds(g * L, L)]
            d_idx = dst_c[pl.ds(g * L, L)]
            a_s = plsc.load_gather(asrc_v, [s_idx])
            a_d = plsc.load_gather(adst_v, [d_idx])
            al = a_s + a_d
            al = jnp.where(al >= 0.0, al, 0.2 * al)
            m = gvec + a_d
            m = jnp.where(m >= 0.0, m, 0.2 * m)
            p = jnp.exp(al - m)
            pos = cb + g * L + lax.iota(jnp.int32, L)
            p = jnp.where(pos < EP, p, 0.0)
            p_v[pl.ds(g * L, L)] = p
        pltpu.make_async_copy(src_hbm.at[pl.ds(0, CH)], rows_c, gsem_c).wait()

        def _scale(j4, _):
            j = j4 * 4
            for u in range(4):
                pj = plsc.load_gather(p_v, [jnp.full((L,), j + u, jnp.int32)])
                for q in range(4):
                    rows_c[j + u, pl.ds(q * L, L)] = (
                        rows_c[j + u, pl.ds(q * L, L)] * pj)
            return 0
        lax.fori_loop(0, CH // 4, _scale, 0)
        pltpu.sync_copy(p_v, den_sh.at[dst_c], add=True)
        for g in range(CH // L):
            dsc_c[pl.ds(g * L, L)] = dst_c[pl.ds(g * L, L)]
        pltpu.async_copy(rows_c, acc_sh.at[dsc_c], ssem_c, add=True)
        nb = base + (k + 2) * CH
        pltpu.async_copy(src_hbm.at[pl.ds(nb, CH)], src_c, si_c)
        pltpu.async_copy(dst_hbm.at[pl.ds(nb, CH)], dst_c, di_c)

    pltpu.sync_copy(src_hbm.at[pl.ds(base, CH)], src_v0)
    pltpu.sync_copy(dst_hbm.at[pl.ds(base, CH)], dst_v0)
    pltpu.async_copy(src_hbm.at[pl.ds(base + CH, CH)], src_v1, si1)
    pltpu.async_copy(dst_hbm.at[pl.ds(base + CH, CH)], dst_v1, di1)
    pltpu.async_copy(h_hbm.at[src_v0], rows_v0, gsem0)

    def _pair_steps(t, _):
        _process(2 * t, bufs[0], bufs[1])
        _process(2 * t + 1, bufs[1], bufs[0])
        return 0
    lax.fori_loop(0, NCHUNK // 2, _pair_steps, 0)

    pltpu.make_async_copy(src_hbm.at[pl.ds(0, CH)], rows_v0, gsem0).wait()
    pltpu.make_async_copy(src_hbm.at[pl.ds(0, CH)], src_v1, si1).wait()
    pltpu.make_async_copy(dst_hbm.at[pl.ds(0, CH)], dst_v1, di1).wait()
    pltpu.make_async_copy(src_hbm.at[pl.ds(0, CH)], rows_v1, ssem1).wait()
    plsc.subcore_barrier()

    for t in range(ROWS_PER_SUB // CH):
        o = sid * ROWS_PER_SUB + t * CH
        pltpu.sync_copy(acc_sh.at[pl.ds(o, CH)], out_hbm.at[cid, pl.ds(o, CH)])
        pltpu.sync_copy(den_sh.at[pl.ds(o, CH)], den_hbm.at[cid, pl.ds(o, CH)])


@functools.cache
def _sc_conv_kernel():
  return pl.kernel(
    _sc_conv_body,
    out_type=(jax.ShapeDtypeStruct((NC, N_ACC, H), jnp.float32),
              jax.ShapeDtypeStruct((NC, N_ACC), jnp.float32)),
    mesh=_sc_mesh(),
    compiler_params=pltpu.CompilerParams(needs_layout_passes=False, use_tc_tiling_on_sc=False),
    scratch_types=[
        pltpu.VMEM((N,), jnp.float32),
        pltpu.VMEM((N,), jnp.float32),
        pltpu.VMEM((L,), jnp.float32),
        pltpu.VMEM((CH,), jnp.int32),
        pltpu.VMEM((CH,), jnp.int32),
        pltpu.VMEM((CH, H), jnp.float32),
        pltpu.VMEM((CH,), jnp.int32),
        pltpu.VMEM((CH,), jnp.int32),
        pltpu.VMEM((CH,), jnp.int32),
        pltpu.VMEM((CH, H), jnp.float32),
        pltpu.VMEM((CH,), jnp.int32),
        pltpu.VMEM((CH,), jnp.float32),
        pltpu.VMEM((CH, H), jnp.float32),
        pltpu.VMEM((CH,), jnp.float32),
        pltpu.VMEM_SHARED((N_ACC, H), jnp.float32),
        pltpu.VMEM_SHARED((N_ACC,), jnp.float32),
        pltpu.SemaphoreType.DMA,
        pltpu.SemaphoreType.DMA,
        pltpu.SemaphoreType.DMA,
        pltpu.SemaphoreType.DMA,
        pltpu.SemaphoreType.DMA,
        pltpu.SemaphoreType.DMA,
        pltpu.SemaphoreType.DMA,
        pltpu.SemaphoreType.DMA,
    ],
  )


def _sc_pair_body(p_hbm, q_hbm, row_hbm, col_hbm, s_hbm,
                  row_v0, col_v0, pbuf0, qbuf0, sbuf0,
                  row_v1, col_v1, pbuf1, qbuf1, sbuf1,
                  gp0, gq0, gp1, gq1, ri0, ci0, ri1, ci1, wsem0, wsem1):
    cid = lax.axis_index("c")
    sid = lax.axis_index("s")
    wid = sid * NC + cid
    base = wid * PER_W2
    bufs = ((row_v0, col_v0, pbuf0, qbuf0, sbuf0, gp0, gq0, ri0, ci0, wsem0),
            (row_v1, col_v1, pbuf1, qbuf1, sbuf1, gp1, gq1, ri1, ci1, wsem1))

    def _process(k, cur, nxt):
        row_c, col_c, pb_c, qb_c, sb_c, gp_c, gq_c, ri_c, ci_c, ws_c = cur
        row_n, col_n, pb_n, qb_n, sb_n, gp_n, gq_n, ri_n, ci_n, ws_n = nxt
        pltpu.make_async_copy(row_hbm.at[pl.ds(0, CH)], row_n, ri_n).wait()
        pltpu.make_async_copy(col_hbm.at[pl.ds(0, CH)], col_n, ci_n).wait()
        pltpu.async_copy(p_hbm.at[row_n], pb_n, gp_n)
        pltpu.async_copy(q_hbm.at[col_n], qb_n, gq_n)
        pltpu.make_async_copy(row_hbm.at[pl.ds(0, CH)], pb_c, gp_c).wait()
        pltpu.make_async_copy(row_hbm.at[pl.ds(0, CH)], qb_c, gq_c).wait()

        @pl.when(k >= 2)
        def _():
            pltpu.make_async_copy(row_hbm.at[pl.ds(0, CH)], sb_c, ws_c).wait()

        def _add(i, _):
            for q in range(4):
                sb_c[i, pl.ds(q * L, L)] = (pb_c[i, pl.ds(q * L, L)]
                                            + qb_c[i, pl.ds(q * L, L)])
            return 0
        lax.fori_loop(0, CH, _add, 0)
        cb = base + k * CH
        pltpu.async_copy(sb_c, s_hbm.at[pl.ds(cb, CH)], ws_c)
        nb = base + (k + 2) * CH
        pltpu.async_copy(row_hbm.at[pl.ds(nb, CH)], row_c, ri_c)
        pltpu.async_copy(col_hbm.at[pl.ds(nb, CH)], col_c, ci_c)

    pltpu.sync_copy(row_hbm.at[pl.ds(base, CH)], row_v0)
    pltpu.sync_copy(col_hbm.at[pl.ds(base, CH)], col_v0)
    pltpu.async_copy(row_hbm.at[pl.ds(base + CH, CH)], row_v1, ri1)
    pltpu.async_copy(col_hbm.at[pl.ds(base + CH, CH)], col_v1, ci1)
    pltpu.async_copy(p_hbm.at[row_v0], pbuf0, gp0)
    pltpu.async_copy(q_hbm.at[col_v0], qbuf0, gq0)

    def _pair_steps(t, _):
        _process(2 * t, bufs[0], bufs[1])
        _process(2 * t + 1, bufs[1], bufs[0])
        return 0
    lax.fori_loop(0, NCHUNK2 // 2, _pair_steps, 0)

    pltpu.make_async_copy(row_hbm.at[pl.ds(0, CH)], pbuf0, gp0).wait()
    pltpu.make_async_copy(row_hbm.at[pl.ds(0, CH)], qbuf0, gq0).wait()
    pltpu.make_async_copy(row_hbm.at[pl.ds(0, CH)], row_v1, ri1).wait()
    pltpu.make_async_copy(col_hbm.at[pl.ds(0, CH)], col_v1, ci1).wait()
    pltpu.make_async_copy(row_hbm.at[pl.ds(0, CH)], sbuf0, wsem0).wait()
    pltpu.make_async_copy(row_hbm.at[pl.ds(0, CH)], sbuf1, wsem1).wait()


@functools.cache
def _sc_pair_kernel():
  return pl.kernel(
    _sc_pair_body,
    out_type=jax.ShapeDtypeStruct((E2_PAD, 2 * H), jnp.float32),
    mesh=_sc_mesh(),
    compiler_params=pltpu.CompilerParams(needs_layout_passes=False, use_tc_tiling_on_sc=False),
    scratch_types=(
        [pltpu.VMEM((CH,), jnp.int32), pltpu.VMEM((CH,), jnp.int32),
         pltpu.VMEM((CH, H), jnp.float32), pltpu.VMEM((CH, H), jnp.float32),
         pltpu.VMEM((CH, 2 * H), jnp.float32)] * 2
        + [pltpu.SemaphoreType.DMA] * 10
    ),
  )


_BLK = 1000
_NBLK = N // _BLK


def _tc1_body(x_ref, w_ref, as_ref, ad_ref,
              h_ref, asrc_ref, adst_ref, gmax_ref):
    i = pl.program_id(0)
    h = jnp.dot(x_ref[...], w_ref[...], preferred_element_type=jnp.float32)
    h_ref[...] = h
    a_s = jnp.sum(h * as_ref[...], axis=1, keepdims=True)
    a_d = jnp.sum(h * ad_ref[...], axis=1, keepdims=True)
    asrc_ref[...] = a_s
    adst_ref[...] = a_d
    bm = jnp.max(a_s)

    bm2 = bm.reshape(1, 1)

    @pl.when(i == 0)
    def _():
        gmax_ref[...] = bm2

    @pl.when(i > 0)
    def _():
        gmax_ref[...] = jnp.maximum(gmax_ref[...], bm2)


def _tc_mid_body(part_ref, den_ref, b_ref, w_ref, as_ref, ad_ref,
                 h_ref, asrc_ref, adst_ref, gmax_ref):
    i = pl.program_id(0)
    agg = part_ref[0] + part_ref[1]
    den = den_ref[:, 0:1] + den_ref[:, 1:2]
    out = agg / (den + 1e-16) + b_ref[...]
    hx = jnp.maximum(out, 0.0)
    h2 = jnp.dot(hx, w_ref[...], preferred_element_type=jnp.float32)
    h_ref[...] = h2
    a_s = jnp.sum(h2 * as_ref[...], axis=1, keepdims=True)
    a_d = jnp.sum(h2 * ad_ref[...], axis=1, keepdims=True)
    asrc_ref[...] = a_s
    adst_ref[...] = a_d
    bm = jnp.max(a_s)

    bm2 = bm.reshape(1, 1)

    @pl.when(i == 0)
    def _():
        gmax_ref[...] = bm2

    @pl.when(i > 0)
    def _():
        gmax_ref[...] = jnp.maximum(gmax_ref[...], bm2)


def _tc3_body(part_ref, den_ref, b_ref, wa_ref, wb_ref, mb_ref,
              p_ref, q_ref):
    agg = part_ref[0] + part_ref[1]
    den = den_ref[:, 0:1] + den_ref[:, 1:2]
    hf = agg / (den + 1e-16) + b_ref[...]
    p_ref[...] = (jnp.dot(hf, wa_ref[...], preferred_element_type=jnp.float32)
                  + mb_ref[...])
    q_ref[...] = jnp.dot(hf, wb_ref[...], preferred_element_type=jnp.float32)


_EBLK = 2000
_NEBLK = EH // _EBLK
_LOG_PRIOR = float(np.log(np.float32(1.0 / 3.0) + np.float32(1e-12)))


def _tc4_body(s_ref, w_ref, b_ref,
              logits_ref, probs_ref, kl_ref, rec_ref):
    i = pl.program_id(0)
    hid = jnp.maximum(s_ref[:, :H], 0.0)
    lf = jnp.dot(hid, w_ref[...], preferred_element_type=jnp.float32) + b_ref[...]
    col = lax.broadcasted_iota(jnp.int32, lf.shape, 1)
    valid = col < 3
    lfm = jnp.where(valid, lf, -jnp.inf)
    m = jnp.max(lfm, axis=1, keepdims=True)
    e = jnp.where(valid, jnp.exp(lf - m), 0.0)
    den = jnp.sum(e, axis=1, keepdims=True)
    p = e / den
    logits_ref[...] = lf[:, 0:3]
    probs_ref[...] = p[:, 0:3]
    lp = jnp.log(jnp.where(valid, p, 1.0))
    kl = jnp.where(valid, p * (lp - _LOG_PRIOR), 0.0)
    p02 = jnp.sum(jnp.where(col == 1, 0.0, e), axis=1, keepdims=True) / den
    rec = jnp.log(p02 + 1e-12)
    kl_s = jnp.sum(kl)
    rec_s = jnp.sum(rec)

    kl_s2 = kl_s.reshape(1, 1)
    rec_s2 = rec_s.reshape(1, 1)

    @pl.when(i == 0)
    def _():
        kl_ref[...] = kl_s2
        rec_ref[...] = rec_s2

    @pl.when(i > 0)
    def _():
        kl_ref[...] = kl_ref[...] + kl_s2
        rec_ref[...] = rec_ref[...] + rec_s2


def _node_stage1(x, W1, att_src1, att_dst1):
    return pl.pallas_call(
        _tc1_body,
        grid=(_NBLK,),
        in_specs=[
            pl.BlockSpec((_BLK, D), lambda i: (i, 0)),
            pl.BlockSpec((D, H), lambda i: (0, 0)),
            pl.BlockSpec((1, H), lambda i: (0, 0)),
            pl.BlockSpec((1, H), lambda i: (0, 0)),
        ],
        out_specs=[
            pl.BlockSpec((_BLK, H), lambda i: (i, 0)),
            pl.BlockSpec((_BLK, 1), lambda i: (i, 0)),
            pl.BlockSpec((_BLK, 1), lambda i: (i, 0)),
            pl.BlockSpec((1, 1), lambda i: (0, 0)),
        ],
        out_shape=[
            jax.ShapeDtypeStruct((N, H), jnp.float32),
            jax.ShapeDtypeStruct((N, 1), jnp.float32),
            jax.ShapeDtypeStruct((N, 1), jnp.float32),
            jax.ShapeDtypeStruct((1, 1), jnp.float32),
        ],
    )(x, W1, att_src1.reshape(1, H), att_dst1.reshape(1, H))


def _node_stage2(part, den_t, bias1, W2, att_src2, att_dst2):
    return pl.pallas_call(
        _tc_mid_body,
        grid=(_NBLK,),
        in_specs=[
            pl.BlockSpec((NC, _BLK, H), lambda i: (0, i, 0)),
            pl.BlockSpec((_BLK, NC), lambda i: (i, 0)),
            pl.BlockSpec((1, H), lambda i: (0, 0)),
            pl.BlockSpec((H, H), lambda i: (0, 0)),
            pl.BlockSpec((1, H), lambda i: (0, 0)),
            pl.BlockSpec((1, H), lambda i: (0, 0)),
        ],
        out_specs=[
            pl.BlockSpec((_BLK, H), lambda i: (i, 0)),
            pl.BlockSpec((_BLK, 1), lambda i: (i, 0)),
            pl.BlockSpec((_BLK, 1), lambda i: (i, 0)),
            pl.BlockSpec((1, 1), lambda i: (0, 0)),
        ],
        out_shape=[
            jax.ShapeDtypeStruct((N, H), jnp.float32),
            jax.ShapeDtypeStruct((N, 1), jnp.float32),
            jax.ShapeDtypeStruct((N, 1), jnp.float32),
            jax.ShapeDtypeStruct((1, 1), jnp.float32),
        ],
    )(part, den_t, bias1.reshape(1, H), W2,
      att_src2.reshape(1, H), att_dst2.reshape(1, H))


def _node_stage3(part, den_t, bias2, mW1, mb1):
    return pl.pallas_call(
        _tc3_body,
        grid=(_NBLK,),
        in_specs=[
            pl.BlockSpec((NC, _BLK, H), lambda i: (0, i, 0)),
            pl.BlockSpec((_BLK, NC), lambda i: (i, 0)),
            pl.BlockSpec((1, H), lambda i: (0, 0)),
            pl.BlockSpec((H, H), lambda i: (0, 0)),
            pl.BlockSpec((H, H), lambda i: (0, 0)),
            pl.BlockSpec((1, H), lambda i: (0, 0)),
        ],
        out_specs=[
            pl.BlockSpec((_BLK, H), lambda i: (i, 0)),
            pl.BlockSpec((_BLK, H), lambda i: (i, 0)),
        ],
        out_shape=[
            jax.ShapeDtypeStruct((N, H), jnp.float32),
            jax.ShapeDtypeStruct((N, H), jnp.float32),
        ],
    )(part, den_t, bias2.reshape(1, H), mW1[:H], mW1[H:], mb1.reshape(1, H))


def _edge_stage(S2, mW2, mb2):
    w_pad = jnp.zeros((H, 128), jnp.float32).at[:, :3].set(mW2)
    b_pad = jnp.zeros((1, 128), jnp.float32).at[0, :3].set(mb2)
    return pl.pallas_call(
        _tc4_body,
        grid=(_NEBLK,),
        in_specs=[
            pl.BlockSpec((_EBLK, 2 * H), lambda i: (i, 0)),
            pl.BlockSpec((H, 128), lambda i: (0, 0)),
            pl.BlockSpec((1, 128), lambda i: (0, 0)),
        ],
        out_specs=[
            pl.BlockSpec((_EBLK, 3), lambda i: (i, 0)),
            pl.BlockSpec((_EBLK, 3), lambda i: (i, 0)),
            pl.BlockSpec((1, 1), lambda i: (0, 0)),
            pl.BlockSpec((1, 1), lambda i: (0, 0)),
        ],
        out_shape=[
            jax.ShapeDtypeStruct((EH, 3), jnp.float32),
            jax.ShapeDtypeStruct((EH, 3), jnp.float32),
            jax.ShapeDtypeStruct((1, 1), jnp.float32),
            jax.ShapeDtypeStruct((1, 1), jnp.float32),
        ],
    )(S2, w_pad, b_pad)


def kernel(x, edge_index, W1, att_src1, att_dst1, bias1,
           W2, att_src2, att_dst2, bias2, mW1, mb1, mW2, mb2):
    src = edge_index[0]
    dst = edge_index[1]
    loop_idx = jnp.arange(N, dtype=jnp.int32)

    n_dummy = E_IDX - EP
    src_full = jnp.concatenate(
        [src, loop_idx, jnp.arange(n_dummy, dtype=jnp.int32) % N])
    dst_full = jnp.concatenate(
        [dst, loop_idx, jnp.zeros((n_dummy,), jnp.int32)])

    n_d2 = E2_IDX - EH
    d2 = jnp.arange(n_d2, dtype=jnp.int32) % N
    rows_q = [jnp.concatenate([src[q * EH:(q + 1) * EH], d2])
              for q in range(NSPLIT)]
    cols_q = [jnp.concatenate([dst[q * EH:(q + 1) * EH], d2])
              for q in range(NSPLIT)]

    h1, as1, ad1, gm1 = _node_stage1(x, W1, att_src1, att_dst1)
    gvec1 = jnp.broadcast_to(gm1.reshape(()), (L,))
    part1, den1 = _sc_conv_kernel()(h1, as1.reshape(N), ad1.reshape(N), gvec1,
                                    src_full, dst_full)

    h2, as2, ad2, gm2 = _node_stage2(part1, den1.T, bias1, W2,
                                     att_src2, att_dst2)
    gvec2 = jnp.broadcast_to(gm2.reshape(()), (L,))
    part2, den2 = _sc_conv_kernel()(h2, as2.reshape(N), ad2.reshape(N), gvec2,
                                    src_full, dst_full)

    P, Q = _node_stage3(part2, den2.T, bias2, mW1, mb1)
    parts = []
    for q in range(NSPLIT):
        s_q = _sc_pair_kernel()(P, Q, rows_q[q], cols_q[q])
        parts.append(_edge_stage(s_q, mW2, mb2))
    logits = jnp.concatenate([t[0] for t in parts], axis=0)
    probs = jnp.concatenate([t[1] for t in parts], axis=0)
    kl_sum = sum(t[2] for t in parts)
    rec_sum = sum(t[3] for t in parts)

    struct_loss = (kl_sum.reshape(()) - rec_sum.reshape(())) / jnp.float32(E)
    return (logits, probs, struct_loss)
